# baseline (speedup 1.0000x reference)
import functools

import jax
import jax.numpy as jnp
from jax import lax
from jax.experimental import pallas as pl
from jax.experimental.pallas import tpu as pltpu
from jax.experimental.pallas import tpu_sc as plsc

D = 768
NH = 12
HD = 64
HH = HD // 2
FH = 512
E = 16
T = 2048
TB = 256
NTB = T // TB
A = T * 2
BLK = 256
NBLK = 32
P = NBLK * BLK
NW = 32
NEG = -1e30


def _attn_body(x_ref, w_ref, b_ref, wqkv_ref, cos_ref, sin_ref, out_ref):
    xb = x_ref[...]
    mu = jnp.mean(xb, axis=1, keepdims=True)
    xc = xb - mu
    var = jnp.mean(xc * xc, axis=1, keepdims=True)
    h = xc * lax.rsqrt(var + 1e-5) * w_ref[...] + b_ref[...]
    qkv = jnp.dot(h.astype(jnp.bfloat16), wqkv_ref[...].astype(jnp.bfloat16),
                  preferred_element_type=jnp.float32)
    cos = cos_ref[...]
    sin = sin_ref[...]
    qe = qkv[:, 0:384] * cos - qkv[:, 384:768] * sin
    qo = qkv[:, 384:768] * cos + qkv[:, 0:384] * sin
    ke = qkv[:, 768:1152] * cos - qkv[:, 1152:1536] * sin
    ko = qkv[:, 1152:1536] * cos + qkv[:, 768:1152] * sin
    b16 = lambda a: a.astype(jnp.bfloat16).astype(jnp.float32)
    qe = b16(qe)
    qo = b16(qo)
    ke = b16(ke)
    ko = b16(ko)
    scale = 1.0 / 8.0
    for n in range(NH):
        qen = qe[:, n * HH:(n + 1) * HH]
        qon = qo[:, n * HH:(n + 1) * HH]
        s = []
        for m in range(NH):
            sm = jnp.sum(
                qen * ke[:, m * HH:(m + 1) * HH] + qon * ko[:, m * HH:(m + 1) * HH],
                axis=1, keepdims=True) * scale
            s.append(sm)
        mx = s[0]
        for m in range(1, NH):
            mx = jnp.maximum(mx, s[m])
        es = [jnp.exp(t_ - mx) for t_ in s]
        den = es[0]
        for m in range(1, NH):
            den = den + es[m]
        rden = 1.0 / den
        acc = None
        for m in range(NH):
            vm = b16(qkv[:, 2 * D + m * HD: 2 * D + (m + 1) * HD])
            c = b16(es[m] * rden) * vm
            acc = c if acc is None else acc + c
        out_ref[n, :, :] = acc


def _run_attn(xf, ln1_w, ln1_b, wqkv_p, cosb, sinb):
    return pl.pallas_call(
        _attn_body,
        grid=(NTB,),
        in_specs=[
            pl.BlockSpec((TB, D), lambda i: (i, 0)),
            pl.BlockSpec((1, D), lambda i: (0, 0)),
            pl.BlockSpec((1, D), lambda i: (0, 0)),
            pl.BlockSpec((D, 3 * D), lambda i: (0, 0)),
            pl.BlockSpec((TB, NH * HH), lambda i: (i, 0)),
            pl.BlockSpec((TB, NH * HH), lambda i: (i, 0)),
        ],
        out_specs=pl.BlockSpec((NH, TB, HD), lambda i: (0, i, 0)),
        out_shape=jax.ShapeDtypeStruct((NH, T, HD), jnp.float32),
    )(xf, ln1_w, ln1_b, wqkv_p, cosb, sinb)


def _mid_body(x_ref, sc_ref, wo_ref, lw_ref, lb_ref, wg_ref, gb_ref,
              xm_ref, h2_ref, idx_ref, gate_ref, pp_ref):
    xm = x_ref[...] + jnp.dot(sc_ref[...].astype(jnp.bfloat16),
                              wo_ref[...].astype(jnp.bfloat16),
                              preferred_element_type=jnp.float32)
    xm_ref[...] = xm
    mu = jnp.mean(xm, axis=1, keepdims=True)
    xc = xm - mu
    var = jnp.mean(xc * xc, axis=1, keepdims=True)
    h2 = xc * lax.rsqrt(var + 1e-5) * lw_ref[...] + lb_ref[...]
    h2_ref[...] = h2
    g = jnp.dot(h2.astype(jnp.bfloat16), wg_ref[...].astype(jnp.bfloat16),
                preferred_element_type=jnp.float32) + gb_ref[...]
    iota = lax.broadcasted_iota(jnp.int32, g.shape, 1)
    big = jnp.int32(10**9)
    v1 = jnp.max(g, axis=1, keepdims=True)
    i1 = jnp.min(jnp.where(g == v1, iota, big), axis=1, keepdims=True)
    gm = jnp.where(iota == i1, NEG, g)
    v2 = jnp.max(gm, axis=1, keepdims=True)
    i2 = jnp.min(jnp.where(gm == v2, iota, big), axis=1, keepdims=True)
    g1 = 1.0 / (1.0 + jnp.exp(v2 - v1))
    g2 = 1.0 - g1
    zi = jnp.zeros_like(i1)
    zf = jnp.zeros_like(g1)
    idx_ref[...] = jnp.concatenate([i1, i2, zi, zi, zi, zi, zi, zi], axis=1)
    gate_ref[...] = jnp.concatenate([g1, g2, zf, zf, zf, zf, zf, zf], axis=1)
    p = jnp.exp(g - v1)
    p = p / jnp.sum(p, axis=1, keepdims=True)
    pp_ref[...] = jnp.sum(p, axis=0, keepdims=True).reshape(1, 1, 128)


def _run_mid(xf, sc, wo, ln2_w, ln2_b, wg_pad, gb_pad):
    return pl.pallas_call(
        _mid_body,
        grid=(NTB,),
        in_specs=[
            pl.BlockSpec((TB, D), lambda i: (i, 0)),
            pl.BlockSpec((TB, D), lambda i: (i, 0)),
            pl.BlockSpec((D, D), lambda i: (0, 0)),
            pl.BlockSpec((1, D), lambda i: (0, 0)),
            pl.BlockSpec((1, D), lambda i: (0, 0)),
            pl.BlockSpec((D, 128), lambda i: (0, 0)),
            pl.BlockSpec((1, 128), lambda i: (0, 0)),
        ],
        out_specs=[
            pl.BlockSpec((TB, D), lambda i: (i, 0)),
            pl.BlockSpec((TB, D), lambda i: (i, 0)),
            pl.BlockSpec((TB, 8), lambda i: (i, 0)),
            pl.BlockSpec((TB, 8), lambda i: (i, 0)),
            pl.BlockSpec((1, 1, 128), lambda i: (i, 0, 0)),
        ],
        out_shape=[
            jax.ShapeDtypeStruct((T, D), jnp.float32),
            jax.ShapeDtypeStruct((T, D), jnp.float32),
            jax.ShapeDtypeStruct((T, 8), jnp.int32),
            jax.ShapeDtypeStruct((T, 8), jnp.float32),
            jax.ShapeDtypeStruct((NTB, 1, 128), jnp.float32),
        ],
    )(xf, sc, wo, ln2_w, ln2_b, wg_pad, gb_pad)


def _route_body(ea_ref, pp_ref, pos_ref, bexp_ref, lb_ref):
    ea = ea_ref[...]
    eiota = lax.broadcasted_iota(jnp.int32, (E, A), 0)
    eq = (jnp.broadcast_to(ea, (E, A)) == eiota).astype(jnp.float32)
    incl = eq
    s = 1
    while s < A:
        incl = incl + jnp.concatenate(
            [jnp.zeros((E, s), jnp.float32), incl[:, :A - s]], axis=1)
        s *= 2
    counts = jnp.sum(eq, axis=1, keepdims=True)
    nb = (counts.astype(jnp.int32) + (BLK - 1)) // BLK
    z = nb
    for s in (1, 2, 4, 8):
        z = z + jnp.concatenate(
            [jnp.zeros((s, 1), jnp.int32), z[:E - s, :]], axis=0)
    off = z - nb
    slotbase = (off * BLK).astype(jnp.float32)
    posf = jnp.sum(eq * (slotbase + incl - 1.0), axis=0, keepdims=True)
    pos_ref[...] = posf.astype(jnp.int32)
    biota = lax.broadcasted_iota(jnp.int32, (E, NBLK), 1)
    cmp = (jnp.broadcast_to(off, (E, NBLK)) <= biota).astype(jnp.float32)
    bexp_ref[...] = jnp.sum(cmp, axis=0, keepdims=True).astype(jnp.int32) - 1
    pm = jnp.sum(pp_ref[...].reshape(NTB, 128), axis=0, keepdims=True)[:, :E]
    ssum = jnp.sum(pm, axis=1, keepdims=True)
    lb = jnp.dot(pm, counts, preferred_element_type=jnp.float32)
    lb_ref[...] = lb * (jnp.float32(E) / jnp.float32(A)) / ssum


def _run_route(ea_row, pp):
    return pl.pallas_call(
        _route_body,
        in_specs=[
            pl.BlockSpec((1, A), lambda: (0, 0)),
            pl.BlockSpec((NTB, 1, 128), lambda: (0, 0, 0)),
        ],
        out_specs=[
            pl.BlockSpec((1, A), lambda: (0, 0)),
            pl.BlockSpec((1, NBLK), lambda: (0, 0)),
            pl.BlockSpec((1, 1), lambda: (0, 0)),
        ],
        out_shape=[
            jax.ShapeDtypeStruct((1, A), jnp.int32),
            jax.ShapeDtypeStruct((1, NBLK), jnp.int32),
            jax.ShapeDtypeStruct((1, 1), jnp.float32),
        ],
    )(ea_row, pp)


def _sc_mesh():
    return plsc.VectorSubcoreMesh(core_axis_name="c", subcore_axis_name="s")


def _sc_wid():
    return lax.axis_index("s") * 2 + lax.axis_index("c")


def _sc_build_inv(pos, tok):
    cpw = A // NW

    @functools.partial(
        pl.kernel, mesh=_sc_mesh(),
        out_type=jax.ShapeDtypeStruct((P,), jnp.int32),
        scratch_types=[
            pltpu.VMEM((cpw,), jnp.int32),
            pltpu.VMEM((cpw,), jnp.int32),
            pltpu.SemaphoreType.DMA,
        ],
    )
    def k(pos_hbm, tok_hbm, inv_hbm, idx_v, val_v, sem):
        base = _sc_wid() * cpw
        pltpu.sync_copy(pos_hbm.at[pl.ds(base, cpw)], idx_v)
        pltpu.sync_copy(tok_hbm.at[pl.ds(base, cpw)], val_v)
        pltpu.async_copy(val_v, inv_hbm.at[idx_v], sem).wait()

    return k(pos, tok)


def _sc_dispatch(inv, h2):
    spw = P // NW
    sub = 64

    @functools.partial(
        pl.kernel, mesh=_sc_mesh(),
        out_type=jax.ShapeDtypeStruct((P, D), jnp.float32),
        scratch_types=[
            pltpu.VMEM((sub,), jnp.int32),
            pltpu.VMEM((sub, D), jnp.float32),
            pltpu.SemaphoreType.DMA,
        ],
    )
    def k(inv_hbm, h2_hbm, x_hbm, idx_v, rows_v, sem):
        base = _sc_wid() * spw
        for j in range(spw // sub):
            pltpu.sync_copy(inv_hbm.at[pl.ds(base + j * sub, sub)], idx_v)
            for i in range(sub // 16):
                c = idx_v[pl.ds(i * 16, 16)]
                idx_v[pl.ds(i * 16, 16)] = jnp.minimum(
                    jnp.maximum(c, 0), jnp.int32(T - 1))
            pltpu.async_copy(h2_hbm.at[idx_v], rows_v, sem).wait()
            pltpu.sync_copy(rows_v, x_hbm.at[pl.ds(base + j * sub, sub)])

    return k(inv, h2)


def _sc_combine_gather(p0, p1, y):
    tpw = T // NW

    @functools.partial(
        pl.kernel, mesh=_sc_mesh(),
        out_type=[
            jax.ShapeDtypeStruct((T, D), jnp.float32),
            jax.ShapeDtypeStruct((T, D), jnp.float32),
        ],
        scratch_types=[
            pltpu.VMEM((tpw,), jnp.int32),
            pltpu.VMEM((tpw, D), jnp.float32),
            pltpu.SemaphoreType.DMA,
        ],
    )
    def k(p0_hbm, p1_hbm, y_hbm, y0_hbm, y1_hbm, idx_v, rows_v, sem):
        base = _sc_wid() * tpw
        pltpu.sync_copy(p0_hbm.at[pl.ds(base, tpw)], idx_v)
        pltpu.async_copy(y_hbm.at[idx_v], rows_v, sem).wait()
        pltpu.sync_copy(rows_v, y0_hbm.at[pl.ds(base, tpw)])
        pltpu.sync_copy(p1_hbm.at[pl.ds(base, tpw)], idx_v)
        pltpu.async_copy(y_hbm.at[idx_v], rows_v, sem).wait()
        pltpu.sync_copy(rows_v, y1_hbm.at[pl.ds(base, tpw)])

    return k(p0, p1, y)


def _ffn_body(bexp_ref, x_ref, w1_ref, w2_ref, o_ref):
    xb = x_ref[...].astype(jnp.bfloat16)
    pre = jnp.dot(xb, w1_ref[0].astype(jnp.bfloat16),
                  preferred_element_type=jnp.float32)
    x1 = pre[:, :FH]
    x2 = pre[:, FH:]
    act = x1 * (1.0 / (1.0 + jnp.exp(-x1))) * x2
    o_ref[...] = jnp.dot(act.astype(jnp.bfloat16), w2_ref[0].astype(jnp.bfloat16),
                         preferred_element_type=jnp.float32)


def _run_ffn(bexp, xs, w1, w2):
    grid_spec = pltpu.PrefetchScalarGridSpec(
        num_scalar_prefetch=1,
        grid=(NBLK,),
        in_specs=[
            pl.BlockSpec((BLK, D), lambda i, b: (i, 0)),
            pl.BlockSpec((1, D, 2 * FH), lambda i, b: (b[i], 0, 0)),
            pl.BlockSpec((1, FH, D), lambda i, b: (b[i], 0, 0)),
        ],
        out_specs=pl.BlockSpec((BLK, D), lambda i, b: (i, 0)),
    )
    return pl.pallas_call(
        _ffn_body,
        grid_spec=grid_spec,
        out_shape=jax.ShapeDtypeStruct((P, D), jnp.float32),
    )(bexp, xs, w1, w2)


def _comb_body(xm_ref, y0_ref, y1_ref, g0_ref, g1_ref, o_ref):
    o_ref[...] = (xm_ref[...] + g0_ref[...] * y0_ref[...]
                  + g1_ref[...] * y1_ref[...])


def _run_comb(xm, y0, y1, g0, g1):
    return pl.pallas_call(
        _comb_body,
        grid=(NTB,),
        in_specs=[
            pl.BlockSpec((TB, D), lambda i: (i, 0)),
            pl.BlockSpec((TB, D), lambda i: (i, 0)),
            pl.BlockSpec((TB, D), lambda i: (i, 0)),
            pl.BlockSpec((TB, 1), lambda i: (i, 0)),
            pl.BlockSpec((TB, 1), lambda i: (i, 0)),
        ],
        out_specs=pl.BlockSpec((TB, D), lambda i: (i, 0)),
        out_shape=jax.ShapeDtypeStruct((T, D), jnp.float32),
    )(xm, y0, y1, g0, g1)


def kernel(x, ln1_w, ln1_b, ln2_w, ln2_b, Wqkv, Wo, Wg, expert_biases, W1, W2):
    xf = x.reshape(T, D)
    cols = jnp.arange(3 * D)
    jj = cols % D
    half = jj >= D // 2
    j2 = jnp.where(half, jj - D // 2, jj)
    n = j2 // HH
    i = j2 % HH
    d_orig = jnp.where(half, 2 * i + 1, 2 * i)
    qk_perm = (cols // D) * D + n * HD + d_orig
    perm = jnp.where(cols < 2 * D, qk_perm, cols)
    wqkv_p = jnp.take(Wqkv, perm, axis=1)

    pos_t = jnp.arange(T, dtype=jnp.float32)[:, None]
    theta = 1.0 / (10000.0 ** (jnp.arange(0, HD, 2, dtype=jnp.float32) / HD))
    ang = pos_t * theta[None, :]
    cosb = jnp.tile(jnp.cos(ang), (1, NH))
    sinb = jnp.tile(jnp.sin(ang), (1, NH))

    ao_t = _run_attn(xf, ln1_w.reshape(1, D), ln1_b.reshape(1, D),
                     wqkv_p, cosb, sinb)
    sc = ao_t.reshape(T, D)

    wg_pad = jnp.zeros((D, 128), jnp.float32).at[:, :E].set(Wg)
    gb_pad = jnp.full((1, 128), NEG, jnp.float32).at[0, :E].set(expert_biases)
    xm, h2, idx8, gate8, pp = _run_mid(
        xf, sc, Wo, ln2_w.reshape(1, D), ln2_b.reshape(1, D), wg_pad, gb_pad)

    ea_row = idx8[:, :2].reshape(1, A)
    pos_row, bexp_row, lb = _run_route(ea_row, pp)

    tok = (jnp.arange(A, dtype=jnp.int32) // 2).astype(jnp.int32)
    inv = _sc_build_inv(pos_row.reshape(A), tok)
    xs = _sc_dispatch(inv, h2)
    ys = _run_ffn(bexp_row.reshape(NBLK), xs, W1, W2)
    pos2 = pos_row.reshape(T, 2)
    y0, y1 = _sc_combine_gather(pos2[:, 0], pos2[:, 1], ys)
    out = _run_comb(xm, y0, y1, gate8[:, 0:1], gate8[:, 1:2])
    return (out.reshape(1, T, D), lb[0, 0])


# speedup vs baseline: 1.6090x; 1.6090x over previous
import functools

import jax
import jax.numpy as jnp
from jax import lax
from jax.experimental import pallas as pl
from jax.experimental.pallas import tpu as pltpu
from jax.experimental.pallas import tpu_sc as plsc

D = 768
NH = 12
HD = 64
HH = HD // 2
FH = 512
E = 16
T = 2048
TB = 256
NTB = T // TB
A = T * 2
BLK = 256
NBLK = 32
P = NBLK * BLK
NW = 32
NEG = -1e30


def _b16(a):
    return a.astype(jnp.bfloat16).astype(jnp.float32)


def _attn_body(x_ref, w_ref, b_ref, wqkv_ref, cos_ref, sin_ref, out_ref):
    xb = x_ref[...]
    mu = jnp.mean(xb, axis=1, keepdims=True)
    xc = xb - mu
    var = jnp.mean(xc * xc, axis=1, keepdims=True)
    h = xc * lax.rsqrt(var + 1e-5) * w_ref[...] + b_ref[...]
    qkv = jnp.dot(h.astype(jnp.bfloat16), wqkv_ref[...].astype(jnp.bfloat16),
                  preferred_element_type=jnp.float32)

    cos = cos_ref[...]
    sin = sin_ref[...]
    even = (lax.broadcasted_iota(jnp.int32, (TB, D), 1) % 2) == 0

    def rope(xq):
        left = jnp.concatenate([xq[:, 1:], xq[:, :1]], axis=1)
        right = jnp.concatenate([xq[:, -1:], xq[:, :-1]], axis=1)
        rot = jnp.where(even, -left, right)
        return xq * cos + rot * sin

    q = _b16(rope(qkv[:, 0:D]))
    k = _b16(rope(qkv[:, D:2 * D]))
    scale = 1.0 / 8.0
    for n in range(NH):
        qn = q[:, n * HD:(n + 1) * HD]
        s = []
        for m in range(NH):
            sm = jnp.sum(qn * k[:, m * HD:(m + 1) * HD],
                         axis=1, keepdims=True) * scale
            s.append(sm)
        mx = s[0]
        for m in range(1, NH):
            mx = jnp.maximum(mx, s[m])
        es = [jnp.exp(t_ - mx) for t_ in s]
        den = es[0]
        for m in range(1, NH):
            den = den + es[m]
        rden = 1.0 / den
        acc = None
        for m in range(NH):
            vm = _b16(qkv[:, 2 * D + m * HD: 2 * D + (m + 1) * HD])
            c = _b16(es[m] * rden) * vm
            acc = c if acc is None else acc + c
        out_ref[n, :, :] = acc


def _run_attn(xf, ln1_w, ln1_b, wqkv, cosb, sinb):
    return pl.pallas_call(
        _attn_body,
        grid=(NTB,),
        in_specs=[
            pl.BlockSpec((TB, D), lambda i: (i, 0)),
            pl.BlockSpec((1, D), lambda i: (0, 0)),
            pl.BlockSpec((1, D), lambda i: (0, 0)),
            pl.BlockSpec((D, 3 * D), lambda i: (0, 0)),
            pl.BlockSpec((TB, D), lambda i: (i, 0)),
            pl.BlockSpec((TB, D), lambda i: (i, 0)),
        ],
        out_specs=pl.BlockSpec((NH, TB, HD), lambda i: (0, i, 0)),
        out_shape=jax.ShapeDtypeStruct((NH, T, HD), jnp.float32),
    )(xf, ln1_w, ln1_b, wqkv, cosb, sinb)


def _mid_body(x_ref, sc_ref, wo_ref, lw_ref, lb_ref, wg_ref, gb_ref,
              xm_ref, h2_ref, idx_ref, gate_ref, pp_ref):
    xm = x_ref[...] + jnp.dot(sc_ref[...].astype(jnp.bfloat16),
                              wo_ref[...].astype(jnp.bfloat16),
                              preferred_element_type=jnp.float32)
    xm_ref[...] = xm
    mu = jnp.mean(xm, axis=1, keepdims=True)
    xc = xm - mu
    var = jnp.mean(xc * xc, axis=1, keepdims=True)
    h2 = xc * lax.rsqrt(var + 1e-5) * lw_ref[...] + lb_ref[...]
    h2_ref[...] = h2
    g = jnp.dot(h2.astype(jnp.bfloat16), wg_ref[...].astype(jnp.bfloat16),
                preferred_element_type=jnp.float32) + gb_ref[...]
    iota = lax.broadcasted_iota(jnp.int32, g.shape, 1)
    big = jnp.int32(10**9)
    v1 = jnp.max(g, axis=1, keepdims=True)
    i1 = jnp.min(jnp.where(g == v1, iota, big), axis=1, keepdims=True)
    gm = jnp.where(iota == i1, NEG, g)
    v2 = jnp.max(gm, axis=1, keepdims=True)
    i2 = jnp.min(jnp.where(gm == v2, iota, big), axis=1, keepdims=True)
    g1 = 1.0 / (1.0 + jnp.exp(v2 - v1))
    g2 = 1.0 - g1
    zi = jnp.zeros_like(i1)
    zf = jnp.zeros_like(g1)
    idx_ref[...] = jnp.concatenate([i1, i2, zi, zi, zi, zi, zi, zi], axis=1)
    gate_ref[...] = jnp.concatenate([g1, g2, zf, zf, zf, zf, zf, zf], axis=1)
    p = jnp.exp(g - v1)
    p = p / jnp.sum(p, axis=1, keepdims=True)
    pp_ref[...] = jnp.sum(p, axis=0, keepdims=True).reshape(1, 1, 128)


def _run_mid(xf, sc, wo, ln2_w, ln2_b, wg_pad, gb_pad):
    return pl.pallas_call(
        _mid_body,
        grid=(NTB,),
        in_specs=[
            pl.BlockSpec((TB, D), lambda i: (i, 0)),
            pl.BlockSpec((TB, D), lambda i: (i, 0)),
            pl.BlockSpec((D, D), lambda i: (0, 0)),
            pl.BlockSpec((1, D), lambda i: (0, 0)),
            pl.BlockSpec((1, D), lambda i: (0, 0)),
            pl.BlockSpec((D, 128), lambda i: (0, 0)),
            pl.BlockSpec((1, 128), lambda i: (0, 0)),
        ],
        out_specs=[
            pl.BlockSpec((TB, D), lambda i: (i, 0)),
            pl.BlockSpec((TB, D), lambda i: (i, 0)),
            pl.BlockSpec((TB, 8), lambda i: (i, 0)),
            pl.BlockSpec((TB, 8), lambda i: (i, 0)),
            pl.BlockSpec((1, 1, 128), lambda i: (i, 0, 0)),
        ],
        out_shape=[
            jax.ShapeDtypeStruct((T, D), jnp.float32),
            jax.ShapeDtypeStruct((T, D), jnp.float32),
            jax.ShapeDtypeStruct((T, 8), jnp.int32),
            jax.ShapeDtypeStruct((T, 8), jnp.float32),
            jax.ShapeDtypeStruct((NTB, 1, 128), jnp.float32),
        ],
    )(xf, sc, wo, ln2_w, ln2_b, wg_pad, gb_pad)


def _route_body(ea_ref, pp_ref, pos_ref, bexp_ref, nact_ref, lb_ref):
    ea = ea_ref[...]
    eiota = lax.broadcasted_iota(jnp.int32, (E, A), 0)
    eq = (jnp.broadcast_to(ea, (E, A)) == eiota).astype(jnp.float32)
    incl = eq
    s = 1
    while s < A:
        incl = incl + jnp.concatenate(
            [jnp.zeros((E, s), jnp.float32), incl[:, :A - s]], axis=1)
        s *= 2
    counts = jnp.sum(eq, axis=1, keepdims=True)
    nb = (counts.astype(jnp.int32) + (BLK - 1)) // BLK
    z = nb
    for s in (1, 2, 4, 8):
        z = z + jnp.concatenate(
            [jnp.zeros((s, 1), jnp.int32), z[:E - s, :]], axis=0)
    off = z - nb
    nact_ref[...] = z[E - 1:E, :]
    slotbase = (off * BLK).astype(jnp.float32)
    posf = jnp.sum(eq * (slotbase + incl - 1.0), axis=0, keepdims=True)
    pos_ref[...] = posf.astype(jnp.int32)
    biota = lax.broadcasted_iota(jnp.int32, (E, NBLK), 1)
    cmp = (jnp.broadcast_to(off, (E, NBLK)) <= biota).astype(jnp.float32)
    bexp_ref[...] = jnp.sum(cmp, axis=0, keepdims=True).astype(jnp.int32) - 1
    pm = jnp.sum(pp_ref[...].reshape(NTB, 128), axis=0, keepdims=True)[:, :E]
    ssum = jnp.sum(pm, axis=1, keepdims=True)
    lb = jnp.dot(pm, counts, preferred_element_type=jnp.float32)
    lb_ref[...] = lb * (jnp.float32(E) / jnp.float32(A)) / ssum


def _run_route(ea_row, pp):
    return pl.pallas_call(
        _route_body,
        in_specs=[
            pl.BlockSpec((1, A), lambda: (0, 0)),
            pl.BlockSpec((NTB, 1, 128), lambda: (0, 0, 0)),
        ],
        out_specs=[
            pl.BlockSpec((1, A), lambda: (0, 0)),
            pl.BlockSpec((1, NBLK), lambda: (0, 0)),
            pl.BlockSpec((1, 1), lambda: (0, 0)),
            pl.BlockSpec((1, 1), lambda: (0, 0)),
        ],
        out_shape=[
            jax.ShapeDtypeStruct((1, A), jnp.int32),
            jax.ShapeDtypeStruct((1, NBLK), jnp.int32),
            jax.ShapeDtypeStruct((1, 1), jnp.int32),
            jax.ShapeDtypeStruct((1, 1), jnp.float32),
        ],
    )(ea_row, pp)


def _sc_mesh():
    return plsc.VectorSubcoreMesh(core_axis_name="c", subcore_axis_name="s")


def _sc_wid():
    return lax.axis_index("s") * 2 + lax.axis_index("c")


def _sc_dispatch(p0, p1, h2):
    tpw = T // NW

    @functools.partial(
        pl.kernel, mesh=_sc_mesh(),
        out_type=jax.ShapeDtypeStruct((P, D), jnp.float32),
        scratch_types=[
            pltpu.VMEM((tpw,), jnp.int32),
            pltpu.VMEM((tpw,), jnp.int32),
            pltpu.VMEM((tpw, D), jnp.float32),
            pltpu.SemaphoreType.DMA,
        ],
    )
    def k(p0_hbm, p1_hbm, h2_hbm, x_hbm, i0_v, i1_v, rows_v, sem):
        base = _sc_wid() * tpw
        pltpu.sync_copy(p0_hbm.at[pl.ds(base, tpw)], i0_v)
        pltpu.sync_copy(p1_hbm.at[pl.ds(base, tpw)], i1_v)
        pltpu.sync_copy(h2_hbm.at[pl.ds(base, tpw)], rows_v)
        c0 = pltpu.async_copy(rows_v, x_hbm.at[i0_v], sem)
        c1 = pltpu.async_copy(rows_v, x_hbm.at[i1_v], sem)
        c0.wait()
        c1.wait()

    return k(p0, p1, h2)


def _sc_combine_gather(p0, p1, y):
    tpw = T // NW

    @functools.partial(
        pl.kernel, mesh=_sc_mesh(),
        out_type=[
            jax.ShapeDtypeStruct((T, D), jnp.float32),
            jax.ShapeDtypeStruct((T, D), jnp.float32),
        ],
        scratch_types=[
            pltpu.VMEM((tpw,), jnp.int32),
            pltpu.VMEM((tpw,), jnp.int32),
            pltpu.VMEM((tpw, D), jnp.float32),
            pltpu.VMEM((tpw, D), jnp.float32),
            pltpu.SemaphoreType.DMA,
        ],
    )
    def k(p0_hbm, p1_hbm, y_hbm, y0_hbm, y1_hbm, i0_v, i1_v, r0_v, r1_v, sem):
        base = _sc_wid() * tpw
        pltpu.sync_copy(p0_hbm.at[pl.ds(base, tpw)], i0_v)
        pltpu.sync_copy(p1_hbm.at[pl.ds(base, tpw)], i1_v)
        c0 = pltpu.async_copy(y_hbm.at[i0_v], r0_v, sem)
        c1 = pltpu.async_copy(y_hbm.at[i1_v], r1_v, sem)
        c0.wait()
        c1.wait()
        pltpu.sync_copy(r0_v, y0_hbm.at[pl.ds(base, tpw)])
        pltpu.sync_copy(r1_v, y1_hbm.at[pl.ds(base, tpw)])

    return k(p0, p1, y)


def _ffn_body(bexp_ref, nact_ref, x_ref, w1_ref, w2_ref, o_ref):
    i = pl.program_id(0)

    @pl.when(i < nact_ref[0])
    def _():
        xb = x_ref[...].astype(jnp.bfloat16)
        pre = jnp.dot(xb, w1_ref[0].astype(jnp.bfloat16),
                      preferred_element_type=jnp.float32)
        x1 = pre[:, :FH]
        x2 = pre[:, FH:]
        act = x1 * (1.0 / (1.0 + jnp.exp(-x1))) * x2
        o_ref[...] = jnp.dot(act.astype(jnp.bfloat16),
                             w2_ref[0].astype(jnp.bfloat16),
                             preferred_element_type=jnp.float32)


def _run_ffn(bexp, nact, xs, w1, w2):
    def wexp(i, b, n):
        return b[jnp.minimum(i, n[0] - 1)]

    grid_spec = pltpu.PrefetchScalarGridSpec(
        num_scalar_prefetch=2,
        grid=(NBLK,),
        in_specs=[
            pl.BlockSpec((BLK, D), lambda i, b, n: (jnp.minimum(i, n[0] - 1), 0)),
            pl.BlockSpec((1, D, 2 * FH), lambda i, b, n: (wexp(i, b, n), 0, 0)),
            pl.BlockSpec((1, FH, D), lambda i, b, n: (wexp(i, b, n), 0, 0)),
        ],
        out_specs=pl.BlockSpec((BLK, D),
                               lambda i, b, n: (jnp.minimum(i, n[0] - 1), 0)),
    )
    return pl.pallas_call(
        _ffn_body,
        grid_spec=grid_spec,
        out_shape=jax.ShapeDtypeStruct((P, D), jnp.float32),
    )(bexp, nact, xs, w1, w2)


def _comb_body(xm_ref, y0_ref, y1_ref, g0_ref, g1_ref, o_ref):
    o_ref[...] = (xm_ref[...] + g0_ref[...] * y0_ref[...]
                  + g1_ref[...] * y1_ref[...])


def _run_comb(xm, y0, y1, g0, g1):
    return pl.pallas_call(
        _comb_body,
        grid=(NTB,),
        in_specs=[
            pl.BlockSpec((TB, D), lambda i: (i, 0)),
            pl.BlockSpec((TB, D), lambda i: (i, 0)),
            pl.BlockSpec((TB, D), lambda i: (i, 0)),
            pl.BlockSpec((TB, 1), lambda i: (i, 0)),
            pl.BlockSpec((TB, 1), lambda i: (i, 0)),
        ],
        out_specs=pl.BlockSpec((TB, D), lambda i: (i, 0)),
        out_shape=jax.ShapeDtypeStruct((T, D), jnp.float32),
    )(xm, y0, y1, g0, g1)


def kernel(x, ln1_w, ln1_b, ln2_w, ln2_b, Wqkv, Wo, Wg, expert_biases, W1, W2):
    xf = x.reshape(T, D)
    theta = 1.0 / (10000.0 ** (jnp.arange(0, HD, 2, dtype=jnp.float32) / HD))
    ang = jnp.arange(T, dtype=jnp.float32)[:, None] * theta[None, :]
    cos2 = jnp.tile(jnp.repeat(jnp.cos(ang), 2, axis=1), (1, NH))
    sin2 = jnp.tile(jnp.repeat(jnp.sin(ang), 2, axis=1), (1, NH))

    ao_t = _run_attn(xf, ln1_w.reshape(1, D), ln1_b.reshape(1, D),
                     Wqkv, cos2, sin2)
    sc = ao_t.reshape(T, D)

    wg_pad = jnp.zeros((D, 128), jnp.float32).at[:, :E].set(Wg)
    gb_pad = jnp.full((1, 128), NEG, jnp.float32).at[0, :E].set(expert_biases)
    xm, h2, idx8, gate8, pp = _run_mid(
        xf, sc, Wo, ln2_w.reshape(1, D), ln2_b.reshape(1, D), wg_pad, gb_pad)

    ea_row = idx8[:, :2].reshape(1, A)
    pos_row, bexp_row, nact, lb = _run_route(ea_row, pp)

    pos2 = pos_row.reshape(T, 2)
    p0 = pos2[:, 0]
    p1 = pos2[:, 1]
    xs = _sc_dispatch(p0, p1, h2)
    ys = _run_ffn(bexp_row.reshape(NBLK), nact.reshape(1), xs, W1, W2)
    y0, y1 = _sc_combine_gather(p0, p1, ys)
    out = _run_comb(xm, y0, y1, gate8[:, 0:1], gate8[:, 1:2])
    return (out.reshape(1, T, D), lb[0, 0])


# speedup vs baseline: 2.2889x; 1.4226x over previous
import functools

import jax
import jax.numpy as jnp
from jax import lax
from jax.experimental import pallas as pl
from jax.experimental.pallas import tpu as pltpu
from jax.experimental.pallas import tpu_sc as plsc

D = 768
NH = 12
HD = 64
HH = HD // 2
FH = 512
E = 16
T = 2048
TB = 256
NTB = T // TB
A = T * 2
BLK = 256
NBLK = 32
P = NBLK * BLK
NW = 32
NEG = -1e30


def _b16(a):
    return a.astype(jnp.bfloat16).astype(jnp.float32)


def _attn_body(x_ref, w_ref, b_ref, wqkv_ref, cos_ref, sin_ref,
               rg_ref, rb_ref, rt_ref, out_ref):
    xb = x_ref[...]
    mu = jnp.mean(xb, axis=1, keepdims=True)
    xc = xb - mu
    var = jnp.mean(xc * xc, axis=1, keepdims=True)
    h = xc * lax.rsqrt(var + 1e-5) * w_ref[...] + b_ref[...]
    qkv = jnp.dot(h.astype(jnp.bfloat16), wqkv_ref[...].astype(jnp.bfloat16),
                  preferred_element_type=jnp.float32)

    cos = cos_ref[...]
    sin = sin_ref[...]
    even = (lax.broadcasted_iota(jnp.int32, (TB, D), 1) % 2) == 0

    def rope(xq):
        left = jnp.concatenate([xq[:, 1:], xq[:, :1]], axis=1)
        right = jnp.concatenate([xq[:, -1:], xq[:, :-1]], axis=1)
        rot = jnp.where(even, -left, right)
        return xq * cos + rot * sin

    q = _b16(rope(qkv[:, 0:D]))
    k = _b16(rope(qkv[:, D:2 * D]))
    rg = rg_ref[...].astype(jnp.bfloat16)
    rb = rb_ref[...]
    del rt_ref
    scale = 1.0 / 8.0
    s = []
    for m in range(NH):
        km = jnp.concatenate([k[:, m * HD:(m + 1) * HD]] * NH, axis=1)
        p = q * km
        hi = p.astype(jnp.bfloat16)
        lo = (p - hi.astype(jnp.float32)).astype(jnp.bfloat16)
        sm = (jnp.dot(hi, rg, preferred_element_type=jnp.float32)
              + jnp.dot(lo, rg, preferred_element_type=jnp.float32))
        s.append(sm * scale)
    mx = s[0]
    for m in range(1, NH):
        mx = jnp.maximum(mx, s[m])
    es = [jnp.exp(t_ - mx) for t_ in s]
    den = es[0]
    for m in range(1, NH):
        den = den + es[m]
    rden = 1.0 / den
    acc = None
    for m in range(NH):
        vm = _b16(qkv[:, 2 * D + m * HD: 2 * D + (m + 1) * HD])
        vt = jnp.concatenate([vm] * NH, axis=1)
        ab = jnp.dot(es[m] * rden, rb, preferred_element_type=jnp.float32)
        c = ab * vt
        acc = c if acc is None else acc + c
    for n in range(NH):
        out_ref[n, :, :] = acc[:, n * HD:(n + 1) * HD]


def _run_attn(xf, ln1_w, ln1_b, wqkv, cosb, sinb, rg, rb, rt):
    return pl.pallas_call(
        _attn_body,
        grid=(NTB,),
        in_specs=[
            pl.BlockSpec((TB, D), lambda i: (i, 0)),
            pl.BlockSpec((1, D), lambda i: (0, 0)),
            pl.BlockSpec((1, D), lambda i: (0, 0)),
            pl.BlockSpec((D, 3 * D), lambda i: (0, 0)),
            pl.BlockSpec((TB, D), lambda i: (i, 0)),
            pl.BlockSpec((TB, D), lambda i: (i, 0)),
            pl.BlockSpec((D, 128), lambda i: (0, 0)),
            pl.BlockSpec((128, D), lambda i: (0, 0)),
            pl.BlockSpec((HD, D), lambda i: (0, 0)),
        ],
        out_specs=pl.BlockSpec((NH, TB, HD), lambda i: (0, i, 0)),
        out_shape=jax.ShapeDtypeStruct((NH, T, HD), jnp.float32),
    )(xf, ln1_w, ln1_b, wqkv, cosb, sinb, rg, rb, rt)


def _mid_body(x_ref, sc_ref, wo_ref, lw_ref, lb_ref, wg_ref, gb_ref,
              xm_ref, h2_ref, idx_ref, gate_ref, pp_ref):
    xm = x_ref[...] + jnp.dot(sc_ref[...].astype(jnp.bfloat16),
                              wo_ref[...].astype(jnp.bfloat16),
                              preferred_element_type=jnp.float32)
    xm_ref[...] = xm
    mu = jnp.mean(xm, axis=1, keepdims=True)
    xc = xm - mu
    var = jnp.mean(xc * xc, axis=1, keepdims=True)
    h2 = xc * lax.rsqrt(var + 1e-5) * lw_ref[...] + lb_ref[...]
    h2_ref[...] = h2
    g = jnp.dot(h2.astype(jnp.bfloat16), wg_ref[...].astype(jnp.bfloat16),
                preferred_element_type=jnp.float32) + gb_ref[...]
    iota = lax.broadcasted_iota(jnp.int32, g.shape, 1)
    big = jnp.int32(10**9)
    v1 = jnp.max(g, axis=1, keepdims=True)
    i1 = jnp.min(jnp.where(g == v1, iota, big), axis=1, keepdims=True)
    gm = jnp.where(iota == i1, NEG, g)
    v2 = jnp.max(gm, axis=1, keepdims=True)
    i2 = jnp.min(jnp.where(gm == v2, iota, big), axis=1, keepdims=True)
    g1 = 1.0 / (1.0 + jnp.exp(v2 - v1))
    g2 = 1.0 - g1
    zi = jnp.zeros_like(i1)
    zf = jnp.zeros_like(g1)
    idx_ref[...] = jnp.concatenate([i1, i2, zi, zi, zi, zi, zi, zi], axis=1)
    gate_ref[...] = jnp.concatenate([g1, g2, zf, zf, zf, zf, zf, zf], axis=1)
    p = jnp.exp(g - v1)
    p = p / jnp.sum(p, axis=1, keepdims=True)
    pp_ref[...] = jnp.sum(p, axis=0, keepdims=True).reshape(1, 1, 128)


def _run_mid(xf, sc, wo, ln2_w, ln2_b, wg_pad, gb_pad):
    return pl.pallas_call(
        _mid_body,
        grid=(NTB,),
        in_specs=[
            pl.BlockSpec((TB, D), lambda i: (i, 0)),
            pl.BlockSpec((TB, D), lambda i: (i, 0)),
            pl.BlockSpec((D, D), lambda i: (0, 0)),
            pl.BlockSpec((1, D), lambda i: (0, 0)),
            pl.BlockSpec((1, D), lambda i: (0, 0)),
            pl.BlockSpec((D, 128), lambda i: (0, 0)),
            pl.BlockSpec((1, 128), lambda i: (0, 0)),
        ],
        out_specs=[
            pl.BlockSpec((TB, D), lambda i: (i, 0)),
            pl.BlockSpec((TB, D), lambda i: (i, 0)),
            pl.BlockSpec((TB, 8), lambda i: (i, 0)),
            pl.BlockSpec((TB, 8), lambda i: (i, 0)),
            pl.BlockSpec((1, 1, 128), lambda i: (i, 0, 0)),
        ],
        out_shape=[
            jax.ShapeDtypeStruct((T, D), jnp.float32),
            jax.ShapeDtypeStruct((T, D), jnp.float32),
            jax.ShapeDtypeStruct((T, 8), jnp.int32),
            jax.ShapeDtypeStruct((T, 8), jnp.float32),
            jax.ShapeDtypeStruct((NTB, 1, 128), jnp.float32),
        ],
    )(xf, sc, wo, ln2_w, ln2_b, wg_pad, gb_pad)


def _route_body(ea_ref, pp_ref, pos_ref, bexp_ref, nact_ref, lb_ref):
    ea = ea_ref[...]
    eiota = lax.broadcasted_iota(jnp.int32, (E, A), 0)
    eq = (jnp.broadcast_to(ea, (E, A)) == eiota).astype(jnp.float32)
    incl = eq
    s = 1
    while s < A:
        incl = incl + jnp.concatenate(
            [jnp.zeros((E, s), jnp.float32), incl[:, :A - s]], axis=1)
        s *= 2
    counts = jnp.sum(eq, axis=1, keepdims=True)
    nb = (counts.astype(jnp.int32) + (BLK - 1)) // BLK
    z = nb
    for s in (1, 2, 4, 8):
        z = z + jnp.concatenate(
            [jnp.zeros((s, 1), jnp.int32), z[:E - s, :]], axis=0)
    off = z - nb
    nact_ref[...] = z[E - 1:E, :]
    slotbase = (off * BLK).astype(jnp.float32)
    posf = jnp.sum(eq * (slotbase + incl - 1.0), axis=0, keepdims=True)
    pos_ref[...] = posf.astype(jnp.int32)
    biota = lax.broadcasted_iota(jnp.int32, (E, NBLK), 1)
    cmp = (jnp.broadcast_to(off, (E, NBLK)) <= biota).astype(jnp.float32)
    bexp_ref[...] = jnp.sum(cmp, axis=0, keepdims=True).astype(jnp.int32) - 1
    pm = jnp.sum(pp_ref[...].reshape(NTB, 128), axis=0, keepdims=True)[:, :E]
    ssum = jnp.sum(pm, axis=1, keepdims=True)
    lb = jnp.dot(pm, counts, preferred_element_type=jnp.float32)
    lb_ref[...] = lb * (jnp.float32(E) / jnp.float32(A)) / ssum


def _run_route(ea_row, pp):
    return pl.pallas_call(
        _route_body,
        in_specs=[
            pl.BlockSpec((1, A), lambda: (0, 0)),
            pl.BlockSpec((NTB, 1, 128), lambda: (0, 0, 0)),
        ],
        out_specs=[
            pl.BlockSpec((1, A), lambda: (0, 0)),
            pl.BlockSpec((1, NBLK), lambda: (0, 0)),
            pl.BlockSpec((1, 1), lambda: (0, 0)),
            pl.BlockSpec((1, 1), lambda: (0, 0)),
        ],
        out_shape=[
            jax.ShapeDtypeStruct((1, A), jnp.int32),
            jax.ShapeDtypeStruct((1, NBLK), jnp.int32),
            jax.ShapeDtypeStruct((1, 1), jnp.int32),
            jax.ShapeDtypeStruct((1, 1), jnp.float32),
        ],
    )(ea_row, pp)


def _sc_mesh():
    return plsc.VectorSubcoreMesh(core_axis_name="c", subcore_axis_name="s")


def _sc_wid():
    return lax.axis_index("s") * 2 + lax.axis_index("c")


def _sc_dispatch(p0, p1, h2):
    tpw = T // NW

    @functools.partial(
        pl.kernel, mesh=_sc_mesh(),
        out_type=jax.ShapeDtypeStruct((P, D), jnp.float32),
        scratch_types=[
            pltpu.VMEM((tpw,), jnp.int32),
            pltpu.VMEM((tpw,), jnp.int32),
            pltpu.VMEM((tpw, D), jnp.float32),
            pltpu.SemaphoreType.DMA,
        ],
    )
    def k(p0_hbm, p1_hbm, h2_hbm, x_hbm, i0_v, i1_v, rows_v, sem):
        base = _sc_wid() * tpw
        pltpu.sync_copy(p0_hbm.at[pl.ds(base, tpw)], i0_v)
        pltpu.sync_copy(p1_hbm.at[pl.ds(base, tpw)], i1_v)
        pltpu.sync_copy(h2_hbm.at[pl.ds(base, tpw)], rows_v)
        c0 = pltpu.async_copy(rows_v, x_hbm.at[i0_v], sem)
        c1 = pltpu.async_copy(rows_v, x_hbm.at[i1_v], sem)
        c0.wait()
        c1.wait()

    return k(p0, p1, h2)


def _sc_combine_gather(p0, p1, y):
    tpw = T // NW

    @functools.partial(
        pl.kernel, mesh=_sc_mesh(),
        out_type=[
            jax.ShapeDtypeStruct((T, D), jnp.float32),
            jax.ShapeDtypeStruct((T, D), jnp.float32),
        ],
        scratch_types=[
            pltpu.VMEM((tpw,), jnp.int32),
            pltpu.VMEM((tpw,), jnp.int32),
            pltpu.VMEM((tpw, D), jnp.float32),
            pltpu.VMEM((tpw, D), jnp.float32),
            pltpu.SemaphoreType.DMA,
        ],
    )
    def k(p0_hbm, p1_hbm, y_hbm, y0_hbm, y1_hbm, i0_v, i1_v, r0_v, r1_v, sem):
        base = _sc_wid() * tpw
        pltpu.sync_copy(p0_hbm.at[pl.ds(base, tpw)], i0_v)
        pltpu.sync_copy(p1_hbm.at[pl.ds(base, tpw)], i1_v)
        c0 = pltpu.async_copy(y_hbm.at[i0_v], r0_v, sem)
        c1 = pltpu.async_copy(y_hbm.at[i1_v], r1_v, sem)
        c0.wait()
        c1.wait()
        pltpu.sync_copy(r0_v, y0_hbm.at[pl.ds(base, tpw)])
        pltpu.sync_copy(r1_v, y1_hbm.at[pl.ds(base, tpw)])

    return k(p0, p1, y)


def _ffn_body(bexp_ref, nact_ref, x_ref, w1_ref, w2_ref, o_ref):
    i = pl.program_id(0)

    @pl.when(i < nact_ref[0])
    def _():
        xb = x_ref[...].astype(jnp.bfloat16)
        pre = jnp.dot(xb, w1_ref[0].astype(jnp.bfloat16),
                      preferred_element_type=jnp.float32)
        x1 = pre[:, :FH]
        x2 = pre[:, FH:]
        act = x1 * (1.0 / (1.0 + jnp.exp(-x1))) * x2
        o_ref[...] = jnp.dot(act.astype(jnp.bfloat16),
                             w2_ref[0].astype(jnp.bfloat16),
                             preferred_element_type=jnp.float32)


def _run_ffn(bexp, nact, xs, w1, w2):
    def wexp(i, b, n):
        return b[jnp.minimum(i, n[0] - 1)]

    grid_spec = pltpu.PrefetchScalarGridSpec(
        num_scalar_prefetch=2,
        grid=(NBLK,),
        in_specs=[
            pl.BlockSpec((BLK, D), lambda i, b, n: (jnp.minimum(i, n[0] - 1), 0)),
            pl.BlockSpec((1, D, 2 * FH), lambda i, b, n: (wexp(i, b, n), 0, 0)),
            pl.BlockSpec((1, FH, D), lambda i, b, n: (wexp(i, b, n), 0, 0)),
        ],
        out_specs=pl.BlockSpec((BLK, D),
                               lambda i, b, n: (jnp.minimum(i, n[0] - 1), 0)),
    )
    return pl.pallas_call(
        _ffn_body,
        grid_spec=grid_spec,
        out_shape=jax.ShapeDtypeStruct((P, D), jnp.float32),
    )(bexp, nact, xs, w1, w2)


def _comb_body(xm_ref, y0_ref, y1_ref, g0_ref, g1_ref, o_ref):
    o_ref[...] = (xm_ref[...] + g0_ref[...] * y0_ref[...]
                  + g1_ref[...] * y1_ref[...])


def _run_comb(xm, y0, y1, g0, g1):
    return pl.pallas_call(
        _comb_body,
        grid=(NTB,),
        in_specs=[
            pl.BlockSpec((TB, D), lambda i: (i, 0)),
            pl.BlockSpec((TB, D), lambda i: (i, 0)),
            pl.BlockSpec((TB, D), lambda i: (i, 0)),
            pl.BlockSpec((TB, 1), lambda i: (i, 0)),
            pl.BlockSpec((TB, 1), lambda i: (i, 0)),
        ],
        out_specs=pl.BlockSpec((TB, D), lambda i: (i, 0)),
        out_shape=jax.ShapeDtypeStruct((T, D), jnp.float32),
    )(xm, y0, y1, g0, g1)


def kernel(x, ln1_w, ln1_b, ln2_w, ln2_b, Wqkv, Wo, Wg, expert_biases, W1, W2):
    xf = x.reshape(T, D)
    theta = 1.0 / (10000.0 ** (jnp.arange(0, HD, 2, dtype=jnp.float32) / HD))
    ang = jnp.arange(T, dtype=jnp.float32)[:, None] * theta[None, :]
    cos2 = jnp.tile(jnp.repeat(jnp.cos(ang), 2, axis=1), (1, NH))
    sin2 = jnp.tile(jnp.repeat(jnp.sin(ang), 2, axis=1), (1, NH))

    jd = jnp.arange(D)
    rg = (jd[:, None] // HD == jnp.arange(128)[None, :]).astype(jnp.float32)
    rb = (jnp.arange(128)[:, None] == jd[None, :] // HD).astype(jnp.float32)
    rt = (jnp.arange(HD)[:, None] == jd[None, :] % HD).astype(jnp.float32)

    ao_t = _run_attn(xf, ln1_w.reshape(1, D), ln1_b.reshape(1, D),
                     Wqkv, cos2, sin2, rg, rb, rt)
    sc = ao_t.reshape(T, D)

    wg_pad = jnp.zeros((D, 128), jnp.float32).at[:, :E].set(Wg)
    gb_pad = jnp.full((1, 128), NEG, jnp.float32).at[0, :E].set(expert_biases)
    xm, h2, idx8, gate8, pp = _run_mid(
        xf, sc, Wo, ln2_w.reshape(1, D), ln2_b.reshape(1, D), wg_pad, gb_pad)

    ea_row = idx8[:, :2].reshape(1, A)
    pos_row, bexp_row, nact, lb = _run_route(ea_row, pp)

    pos2 = pos_row.reshape(T, 2)
    p0 = pos2[:, 0]
    p1 = pos2[:, 1]
    xs = _sc_dispatch(p0, p1, h2)
    ys = _run_ffn(bexp_row.reshape(NBLK), nact.reshape(1), xs, W1, W2)
    y0, y1 = _sc_combine_gather(p0, p1, ys)
    out = _run_comb(xm, y0, y1, gate8[:, 0:1], gate8[:, 1:2])
    return (out.reshape(1, T, D), lb[0, 0])


# speedup vs baseline: 2.2907x; 1.0008x over previous
import functools

import jax
import jax.numpy as jnp
from jax import lax
from jax.experimental import pallas as pl
from jax.experimental.pallas import tpu as pltpu
from jax.experimental.pallas import tpu_sc as plsc

D = 768
NH = 12
HD = 64
HH = HD // 2
FH = 512
E = 16
T = 2048
TB = 256
NTB = T // TB
A = T * 2
BLK = 256
NBLK = 32
P = NBLK * BLK
NW = 32
NEG = -1e30


def _b16(a):
    return a.astype(jnp.bfloat16).astype(jnp.float32)


def _attn_body(x_ref, w_ref, b_ref, wqkv_ref, cos_ref, sin_ref,
               rg_ref, rb_ref, rt_ref, out_ref):
    xb = x_ref[...]
    mu = jnp.mean(xb, axis=1, keepdims=True)
    xc = xb - mu
    var = jnp.mean(xc * xc, axis=1, keepdims=True)
    h = xc * lax.rsqrt(var + 1e-5) * w_ref[...] + b_ref[...]
    qkv = jnp.dot(h.astype(jnp.bfloat16), wqkv_ref[...].astype(jnp.bfloat16),
                  preferred_element_type=jnp.float32)

    cos = cos_ref[...]
    sin = sin_ref[...]
    even = (lax.broadcasted_iota(jnp.int32, (TB, D), 1) % 2) == 0

    def rope(xq):
        left = jnp.concatenate([xq[:, 1:], xq[:, :1]], axis=1)
        right = jnp.concatenate([xq[:, -1:], xq[:, :-1]], axis=1)
        rot = jnp.where(even, -left, right)
        return xq * cos + rot * sin

    q = _b16(rope(qkv[:, 0:D]))
    k = _b16(rope(qkv[:, D:2 * D]))
    rg = rg_ref[...].astype(jnp.bfloat16)
    rb = rb_ref[...]
    del rt_ref
    scale = 1.0 / 8.0
    s = []
    for m in range(NH):
        km = jnp.concatenate([k[:, m * HD:(m + 1) * HD]] * NH, axis=1)
        p = q * km
        hi = p.astype(jnp.bfloat16)
        lo = (p - hi.astype(jnp.float32)).astype(jnp.bfloat16)
        sm = (jnp.dot(hi, rg, preferred_element_type=jnp.float32)
              + jnp.dot(lo, rg, preferred_element_type=jnp.float32))
        s.append(sm * scale)
    mx = s[0]
    for m in range(1, NH):
        mx = jnp.maximum(mx, s[m])
    es = [jnp.exp(t_ - mx) for t_ in s]
    den = es[0]
    for m in range(1, NH):
        den = den + es[m]
    acc = None
    for m in range(NH):
        vm = _b16(qkv[:, 2 * D + m * HD: 2 * D + (m + 1) * HD])
        vt = jnp.concatenate([vm] * NH, axis=1)
        ab = jnp.dot(es[m] / den, rb, preferred_element_type=jnp.float32)
        c = ab * vt
        acc = c if acc is None else acc + c
    for n in range(NH):
        out_ref[n, :, :] = acc[:, n * HD:(n + 1) * HD]


def _run_attn(xf, ln1_w, ln1_b, wqkv, cosb, sinb, rg, rb, rt):
    return pl.pallas_call(
        _attn_body,
        grid=(NTB,),
        in_specs=[
            pl.BlockSpec((TB, D), lambda i: (i, 0)),
            pl.BlockSpec((1, D), lambda i: (0, 0)),
            pl.BlockSpec((1, D), lambda i: (0, 0)),
            pl.BlockSpec((D, 3 * D), lambda i: (0, 0)),
            pl.BlockSpec((TB, D), lambda i: (i, 0)),
            pl.BlockSpec((TB, D), lambda i: (i, 0)),
            pl.BlockSpec((D, 128), lambda i: (0, 0)),
            pl.BlockSpec((128, D), lambda i: (0, 0)),
            pl.BlockSpec((HD, D), lambda i: (0, 0)),
        ],
        out_specs=pl.BlockSpec((NH, TB, HD), lambda i: (0, i, 0)),
        out_shape=jax.ShapeDtypeStruct((NH, T, HD), jnp.float32),
    )(xf, ln1_w, ln1_b, wqkv, cosb, sinb, rg, rb, rt)


def _mid_body(x_ref, sc_ref, wo_ref, lw_ref, lb_ref, wg_ref, gb_ref,
              xm_ref, h2b_ref, idx_ref, gate_ref, pp_ref):
    xm = x_ref[...] + jnp.dot(sc_ref[...].astype(jnp.bfloat16),
                              wo_ref[...].astype(jnp.bfloat16),
                              preferred_element_type=jnp.float32)
    xm_ref[...] = xm
    mu = jnp.mean(xm, axis=1, keepdims=True)
    xc = xm - mu
    var = jnp.mean(xc * xc, axis=1, keepdims=True)
    h2 = xc * lax.rsqrt(var + 1e-5) * lw_ref[...] + lb_ref[...]
    h2b_ref[...] = h2
    g = jnp.dot(h2.astype(jnp.bfloat16), wg_ref[...].astype(jnp.bfloat16),
                preferred_element_type=jnp.float32) + gb_ref[...]
    iota = lax.broadcasted_iota(jnp.int32, g.shape, 1)
    big = jnp.int32(10**9)
    v1 = jnp.max(g, axis=1, keepdims=True)
    i1 = jnp.min(jnp.where(g == v1, iota, big), axis=1, keepdims=True)
    gm = jnp.where(iota == i1, NEG, g)
    v2 = jnp.max(gm, axis=1, keepdims=True)
    i2 = jnp.min(jnp.where(gm == v2, iota, big), axis=1, keepdims=True)
    ex = jnp.exp(v2 - v1)
    den2 = 1.0 + ex
    g1 = 1.0 / den2
    g2 = ex / den2
    zi = jnp.zeros_like(i1)
    zf = jnp.zeros_like(g1)
    idx_ref[...] = jnp.concatenate([i1, i2, zi, zi, zi, zi, zi, zi], axis=1)
    gate_ref[...] = jnp.concatenate([g1, g2, zf, zf, zf, zf, zf, zf], axis=1)
    p = jnp.exp(g - v1)
    p = p / jnp.sum(p, axis=1, keepdims=True)
    pp_ref[...] = jnp.sum(p, axis=0, keepdims=True).reshape(1, 1, 128)


def _run_mid(xf, sc, wo, ln2_w, ln2_b, wg_pad, gb_pad):
    return pl.pallas_call(
        _mid_body,
        grid=(NTB,),
        in_specs=[
            pl.BlockSpec((TB, D), lambda i: (i, 0)),
            pl.BlockSpec((TB, D), lambda i: (i, 0)),
            pl.BlockSpec((D, D), lambda i: (0, 0)),
            pl.BlockSpec((1, D), lambda i: (0, 0)),
            pl.BlockSpec((1, D), lambda i: (0, 0)),
            pl.BlockSpec((D, 128), lambda i: (0, 0)),
            pl.BlockSpec((1, 128), lambda i: (0, 0)),
        ],
        out_specs=[
            pl.BlockSpec((TB, D), lambda i: (i, 0)),
            pl.BlockSpec((TB, D), lambda i: (i, 0)),
            pl.BlockSpec((TB, 8), lambda i: (i, 0)),
            pl.BlockSpec((TB, 8), lambda i: (i, 0)),
            pl.BlockSpec((1, 1, 128), lambda i: (i, 0, 0)),
        ],
        out_shape=[
            jax.ShapeDtypeStruct((T, D), jnp.float32),
            jax.ShapeDtypeStruct((T, D), jnp.float32),
            jax.ShapeDtypeStruct((T, 8), jnp.int32),
            jax.ShapeDtypeStruct((T, 8), jnp.float32),
            jax.ShapeDtypeStruct((NTB, 1, 128), jnp.float32),
        ],
    )(xf, sc, wo, ln2_w, ln2_b, wg_pad, gb_pad)


def _route_body(ea_ref, pp_ref, pos_ref, bexp_ref, nact_ref, lb_ref):
    ea = ea_ref[...]
    eiota = lax.broadcasted_iota(jnp.int32, (E, A), 0)
    eq = (jnp.broadcast_to(ea, (E, A)) == eiota).astype(jnp.float32)
    incl = eq
    s = 1
    while s < A:
        incl = incl + jnp.concatenate(
            [jnp.zeros((E, s), jnp.float32), incl[:, :A - s]], axis=1)
        s *= 2
    counts = jnp.sum(eq, axis=1, keepdims=True)
    nb = (counts.astype(jnp.int32) + (BLK - 1)) // BLK
    z = nb
    for s in (1, 2, 4, 8):
        z = z + jnp.concatenate(
            [jnp.zeros((s, 1), jnp.int32), z[:E - s, :]], axis=0)
    off = z - nb
    nact_ref[...] = z[E - 1:E, :]
    slotbase = (off * BLK).astype(jnp.float32)
    posf = jnp.sum(eq * (slotbase + incl - 1.0), axis=0, keepdims=True)
    pos_ref[...] = posf.astype(jnp.int32)
    biota = lax.broadcasted_iota(jnp.int32, (E, NBLK), 1)
    cmp = (jnp.broadcast_to(off, (E, NBLK)) <= biota).astype(jnp.float32)
    bexp_ref[...] = jnp.sum(cmp, axis=0, keepdims=True).astype(jnp.int32) - 1
    pm = jnp.sum(pp_ref[...].reshape(NTB, 128), axis=0, keepdims=True)[:, :E]
    ssum = jnp.sum(pm, axis=1, keepdims=True)
    lb = jnp.dot(pm, counts, preferred_element_type=jnp.float32)
    lb_ref[...] = lb * (jnp.float32(E) / jnp.float32(A)) / ssum


def _run_route(ea_row, pp):
    return pl.pallas_call(
        _route_body,
        in_specs=[
            pl.BlockSpec((1, A), lambda: (0, 0)),
            pl.BlockSpec((NTB, 1, 128), lambda: (0, 0, 0)),
        ],
        out_specs=[
            pl.BlockSpec((1, A), lambda: (0, 0)),
            pl.BlockSpec((1, NBLK), lambda: (0, 0)),
            pl.BlockSpec((1, 1), lambda: (0, 0)),
            pl.BlockSpec((1, 1), lambda: (0, 0)),
        ],
        out_shape=[
            jax.ShapeDtypeStruct((1, A), jnp.int32),
            jax.ShapeDtypeStruct((1, NBLK), jnp.int32),
            jax.ShapeDtypeStruct((1, 1), jnp.int32),
            jax.ShapeDtypeStruct((1, 1), jnp.float32),
        ],
    )(ea_row, pp)


def _sc_mesh():
    return plsc.VectorSubcoreMesh(core_axis_name="c", subcore_axis_name="s")


def _sc_wid():
    return lax.axis_index("s") * 2 + lax.axis_index("c")


def _sc_dispatch(p0, p1, h2):
    tpw = T // NW

    @functools.partial(
        pl.kernel, mesh=_sc_mesh(),
        out_type=jax.ShapeDtypeStruct((P, D), jnp.float32),
        scratch_types=[
            pltpu.VMEM((tpw,), jnp.int32),
            pltpu.VMEM((tpw,), jnp.int32),
            pltpu.VMEM((tpw, D), jnp.float32),
            pltpu.SemaphoreType.DMA,
        ],
    )
    def k(p0_hbm, p1_hbm, h2_hbm, x_hbm, i0_v, i1_v, rows_v, sem):
        base = _sc_wid() * tpw
        pltpu.sync_copy(p0_hbm.at[pl.ds(base, tpw)], i0_v)
        pltpu.sync_copy(p1_hbm.at[pl.ds(base, tpw)], i1_v)
        pltpu.sync_copy(h2_hbm.at[pl.ds(base, tpw)], rows_v)
        c0 = pltpu.async_copy(rows_v, x_hbm.at[i0_v], sem)
        c1 = pltpu.async_copy(rows_v, x_hbm.at[i1_v], sem)
        c0.wait()
        c1.wait()

    return k(p0, p1, h2)


def _sc_combine_gather(p0, p1, y):
    tpw = T // NW

    @functools.partial(
        pl.kernel, mesh=_sc_mesh(),
        out_type=[
            jax.ShapeDtypeStruct((T, D), jnp.float32),
            jax.ShapeDtypeStruct((T, D), jnp.float32),
        ],
        scratch_types=[
            pltpu.VMEM((tpw,), jnp.int32),
            pltpu.VMEM((tpw,), jnp.int32),
            pltpu.VMEM((tpw, D), jnp.float32),
            pltpu.VMEM((tpw, D), jnp.float32),
            pltpu.SemaphoreType.DMA,
        ],
    )
    def k(p0_hbm, p1_hbm, y_hbm, y0_hbm, y1_hbm, i0_v, i1_v, r0_v, r1_v, sem):
        base = _sc_wid() * tpw
        pltpu.sync_copy(p0_hbm.at[pl.ds(base, tpw)], i0_v)
        pltpu.sync_copy(p1_hbm.at[pl.ds(base, tpw)], i1_v)
        c0 = pltpu.async_copy(y_hbm.at[i0_v], r0_v, sem)
        c1 = pltpu.async_copy(y_hbm.at[i1_v], r1_v, sem)
        c0.wait()
        c1.wait()
        pltpu.sync_copy(r0_v, y0_hbm.at[pl.ds(base, tpw)])
        pltpu.sync_copy(r1_v, y1_hbm.at[pl.ds(base, tpw)])

    return k(p0, p1, y)


def _ffn_body(bexp_ref, nact_ref, x_ref, w1_ref, w2_ref, o_ref):
    i = pl.program_id(0)

    @pl.when(i < nact_ref[0])
    def _():
        xb = x_ref[...].astype(jnp.bfloat16)
        pre = jnp.dot(xb, w1_ref[0].astype(jnp.bfloat16),
                      preferred_element_type=jnp.float32)
        x1 = pre[:, :FH]
        x2 = pre[:, FH:]
        act = x1 * (1.0 / (1.0 + jnp.exp(-x1))) * x2
        o_ref[...] = jnp.dot(act.astype(jnp.bfloat16),
                             w2_ref[0].astype(jnp.bfloat16),
                             preferred_element_type=jnp.float32)


def _run_ffn(bexp, nact, xs, w1, w2):
    def wexp(i, b, n):
        return b[jnp.minimum(i, n[0] - 1)]

    grid_spec = pltpu.PrefetchScalarGridSpec(
        num_scalar_prefetch=2,
        grid=(NBLK,),
        in_specs=[
            pl.BlockSpec((BLK, D), lambda i, b, n: (jnp.minimum(i, n[0] - 1), 0)),
            pl.BlockSpec((1, D, 2 * FH), lambda i, b, n: (wexp(i, b, n), 0, 0)),
            pl.BlockSpec((1, FH, D), lambda i, b, n: (wexp(i, b, n), 0, 0)),
        ],
        out_specs=pl.BlockSpec((BLK, D),
                               lambda i, b, n: (jnp.minimum(i, n[0] - 1), 0)),
    )
    return pl.pallas_call(
        _ffn_body,
        grid_spec=grid_spec,
        out_shape=jax.ShapeDtypeStruct((P, D), jnp.float32),
    )(bexp, nact, xs, w1, w2)


def _comb_body(xm_ref, y0_ref, y1_ref, g0_ref, g1_ref, o_ref):
    o_ref[...] = (xm_ref[...]
                  + _b16(g0_ref[...]) * _b16(y0_ref[...])
                  + _b16(g1_ref[...]) * _b16(y1_ref[...]))


def _run_comb(xm, y0, y1, g0, g1):
    return pl.pallas_call(
        _comb_body,
        grid=(NTB,),
        in_specs=[
            pl.BlockSpec((TB, D), lambda i: (i, 0)),
            pl.BlockSpec((TB, D), lambda i: (i, 0)),
            pl.BlockSpec((TB, D), lambda i: (i, 0)),
            pl.BlockSpec((TB, 1), lambda i: (i, 0)),
            pl.BlockSpec((TB, 1), lambda i: (i, 0)),
        ],
        out_specs=pl.BlockSpec((TB, D), lambda i: (i, 0)),
        out_shape=jax.ShapeDtypeStruct((T, D), jnp.float32),
    )(xm, y0, y1, g0, g1)


def kernel(x, ln1_w, ln1_b, ln2_w, ln2_b, Wqkv, Wo, Wg, expert_biases, W1, W2):
    xf = x.reshape(T, D)
    theta = 1.0 / (10000.0 ** (jnp.arange(0, HD, 2, dtype=jnp.float32) / HD))
    ang = jnp.arange(T, dtype=jnp.float32)[:, None] * theta[None, :]
    cos2 = jnp.tile(jnp.repeat(jnp.cos(ang), 2, axis=1), (1, NH))
    sin2 = jnp.tile(jnp.repeat(jnp.sin(ang), 2, axis=1), (1, NH))

    jd = jnp.arange(D)
    rg = (jd[:, None] // HD == jnp.arange(128)[None, :]).astype(jnp.float32)
    rb = (jnp.arange(128)[:, None] == jd[None, :] // HD).astype(jnp.float32)
    rt = (jnp.arange(HD)[:, None] == jd[None, :] % HD).astype(jnp.float32)

    ao_t = _run_attn(xf, ln1_w.reshape(1, D), ln1_b.reshape(1, D),
                     Wqkv, cos2, sin2, rg, rb, rt)
    sc = ao_t.reshape(T, D)

    wg_pad = jnp.zeros((D, 128), jnp.float32).at[:, :E].set(Wg)
    gb_pad = jnp.full((1, 128), NEG, jnp.float32).at[0, :E].set(expert_biases)
    xm, h2, idx8, gate8, pp = _run_mid(
        xf, sc, Wo, ln2_w.reshape(1, D), ln2_b.reshape(1, D), wg_pad, gb_pad)

    ea_row = idx8[:, :2].reshape(1, A)
    pos_row, bexp_row, nact, lb = _run_route(ea_row, pp)

    pos2 = pos_row.reshape(T, 2)
    p0 = pos2[:, 0]
    p1 = pos2[:, 1]
    xs = _sc_dispatch(p0, p1, h2)
    ys = _run_ffn(bexp_row.reshape(NBLK), nact.reshape(1), xs, W1, W2)
    y0, y1 = _sc_combine_gather(p0, p1, ys)
    out = _run_comb(xm, y0, y1, gate8[:, 0:1], gate8[:, 1:2])
    return (out.reshape(1, T, D), lb[0, 0])


# speedup vs baseline: 2.2932x; 1.0011x over previous
import functools

import jax
import jax.numpy as jnp
from jax import lax
from jax.experimental import pallas as pl
from jax.experimental.pallas import tpu as pltpu
from jax.experimental.pallas import tpu_sc as plsc

D = 768
NH = 12
HD = 64
HH = HD // 2
FH = 512
E = 16
T = 2048
TB = 256
NTB = T // TB
A = T * 2
BLK = 256
NBLK = 32
P = NBLK * BLK
NW = 32
NEG = -1e30


def _b16(a):
    return a.astype(jnp.bfloat16).astype(jnp.float32)


def _attn_body(x_ref, w_ref, b_ref, wqkv_ref, cos_ref, sin_ref,
               rg_ref, rb_ref, rt_ref, out_ref):
    xb = x_ref[...]
    mu = jnp.mean(xb, axis=1, keepdims=True)
    xc = xb - mu
    var = jnp.mean(xc * xc, axis=1, keepdims=True)
    h = xc * lax.rsqrt(var + 1e-5) * w_ref[...] + b_ref[...]
    qkv = jnp.dot(h.astype(jnp.bfloat16), wqkv_ref[...].astype(jnp.bfloat16),
                  preferred_element_type=jnp.float32)

    cos = cos_ref[...]
    sin = sin_ref[...]
    even = (lax.broadcasted_iota(jnp.int32, (TB, D), 1) % 2) == 0

    def rope(xq):
        left = jnp.concatenate([xq[:, 1:], xq[:, :1]], axis=1)
        right = jnp.concatenate([xq[:, -1:], xq[:, :-1]], axis=1)
        rot = jnp.where(even, -left, right)
        return xq * cos + rot * sin

    q = _b16(rope(qkv[:, 0:D]))
    k = _b16(rope(qkv[:, D:2 * D]))
    rg = rg_ref[...].astype(jnp.bfloat16)
    rb = rb_ref[...]
    del rt_ref
    scale = 1.0 / 8.0
    s = []
    for m in range(NH):
        km = jnp.concatenate([k[:, m * HD:(m + 1) * HD]] * NH, axis=1)
        p = q * km
        hi = p.astype(jnp.bfloat16)
        lo = (p - hi.astype(jnp.float32)).astype(jnp.bfloat16)
        sm = (jnp.dot(hi, rg, preferred_element_type=jnp.float32)
              + jnp.dot(lo, rg, preferred_element_type=jnp.float32))
        s.append(sm * scale)
    mx = s[0]
    for m in range(1, NH):
        mx = jnp.maximum(mx, s[m])
    es = [jnp.exp(t_ - mx) for t_ in s]
    den = es[0]
    for m in range(1, NH):
        den = den + es[m]
    acc = None
    for m in range(NH):
        vm = _b16(qkv[:, 2 * D + m * HD: 2 * D + (m + 1) * HD])
        vt = jnp.concatenate([vm] * NH, axis=1)
        ab = jnp.dot(es[m] / den, rb, preferred_element_type=jnp.float32)
        c = ab * vt
        acc = c if acc is None else acc + c
    for n in range(NH):
        out_ref[n, :, :] = acc[:, n * HD:(n + 1) * HD]


def _run_attn(xf, ln1_w, ln1_b, wqkv, cosb, sinb, rg, rb, rt):
    return pl.pallas_call(
        _attn_body,
        grid=(NTB,),
        in_specs=[
            pl.BlockSpec((TB, D), lambda i: (i, 0)),
            pl.BlockSpec((1, D), lambda i: (0, 0)),
            pl.BlockSpec((1, D), lambda i: (0, 0)),
            pl.BlockSpec((D, 3 * D), lambda i: (0, 0)),
            pl.BlockSpec((TB, D), lambda i: (i, 0)),
            pl.BlockSpec((TB, D), lambda i: (i, 0)),
            pl.BlockSpec((D, 128), lambda i: (0, 0)),
            pl.BlockSpec((128, D), lambda i: (0, 0)),
            pl.BlockSpec((HD, D), lambda i: (0, 0)),
        ],
        out_specs=pl.BlockSpec((NH, TB, HD), lambda i: (0, i, 0)),
        out_shape=jax.ShapeDtypeStruct((NH, T, HD), jnp.float32),
    )(xf, ln1_w, ln1_b, wqkv, cosb, sinb, rg, rb, rt)


def _mid_body(x_ref, sc_ref, wo_ref, lw_ref, lb_ref, wg_ref, gb_ref,
              xm_ref, h2b_ref, gate_ref, p0_ref, p1_ref, bexp_ref, nact_ref,
              lb_out_ref, i1_scr, i2_scr, pp_scr):
    i = pl.program_id(0)
    xm = x_ref[...] + jnp.dot(sc_ref[...].astype(jnp.bfloat16),
                              wo_ref[...].astype(jnp.bfloat16),
                              preferred_element_type=jnp.float32)
    xm_ref[...] = xm
    mu = jnp.mean(xm, axis=1, keepdims=True)
    xc = xm - mu
    var = jnp.mean(xc * xc, axis=1, keepdims=True)
    h2 = xc * lax.rsqrt(var + 1e-5) * lw_ref[...] + lb_ref[...]
    h2b_ref[...] = h2
    g = jnp.dot(h2.astype(jnp.bfloat16), wg_ref[...].astype(jnp.bfloat16),
                preferred_element_type=jnp.float32) + gb_ref[...]
    iota = lax.broadcasted_iota(jnp.int32, g.shape, 1)
    big = jnp.int32(10**9)
    v1 = jnp.max(g, axis=1, keepdims=True)
    i1 = jnp.min(jnp.where(g == v1, iota, big), axis=1, keepdims=True)
    gm = jnp.where(iota == i1, NEG, g)
    v2 = jnp.max(gm, axis=1, keepdims=True)
    i2 = jnp.min(jnp.where(gm == v2, iota, big), axis=1, keepdims=True)
    ex = jnp.exp(v2 - v1)
    den2 = 1.0 + ex
    g1 = 1.0 / den2
    g2 = ex / den2
    zf = jnp.zeros_like(g1)
    gate_ref[...] = jnp.concatenate([g1, g2, zf, zf, zf, zf, zf, zf], axis=1)
    i1_scr[pl.ds(i * TB, TB), :] = i1
    i2_scr[pl.ds(i * TB, TB), :] = i2
    p = jnp.exp(g - v1)
    p = p / jnp.sum(p, axis=1, keepdims=True)
    psum = jnp.sum(p, axis=0, keepdims=True)

    @pl.when(i == 0)
    def _():
        pp_scr[...] = psum

    @pl.when(i > 0)
    def _():
        pp_scr[...] = pp_scr[...] + psum

    @pl.when(i == NTB - 1)
    def _():
        lane = lax.broadcasted_iota(jnp.int32, (T, E), 1)
        i1a = i1_scr[...]
        i2a = i2_scr[...]
        eq0 = (jnp.broadcast_to(i1a, (T, E)) == lane).astype(jnp.float32)
        eq1 = (jnp.broadcast_to(i2a, (T, E)) == lane).astype(jnp.float32)
        both = eq0 + eq1
        c = both
        s = 1
        while s < T:
            c = c + jnp.concatenate(
                [jnp.zeros((s, E), jnp.float32), c[:T - s]], axis=0)
            s *= 2
        excl = c - both
        counts = jnp.sum(both, axis=0, keepdims=True)
        nb = (counts.astype(jnp.int32) + (BLK - 1)) // BLK
        z = nb
        for s in (1, 2, 4, 8):
            z = z + jnp.concatenate(
                [jnp.zeros((1, s), jnp.int32), z[:, :E - s]], axis=1)
        off = z - nb
        nact_ref[...] = z[:, E - 1:E]
        slotbase = jnp.broadcast_to((off * BLK).astype(jnp.float32), (T, E))
        p0f = jnp.sum(eq0 * (slotbase + excl), axis=1, keepdims=True)
        p1f = jnp.sum(eq1 * (slotbase + excl + eq0), axis=1, keepdims=True)
        p0_ref[...] = p0f.astype(jnp.int32)
        p1_ref[...] = p1f.astype(jnp.int32)
        biota = lax.broadcasted_iota(jnp.int32, (1, NBLK), 1)
        acc = jnp.full((1, NBLK), -1, jnp.int32)
        for e in range(E):
            acc = acc + (jnp.broadcast_to(off[:, e:e + 1], (1, NBLK))
                         <= biota).astype(jnp.int32)
        bexp_ref[...] = acc
        pm = pp_scr[...][:, :E]
        ssum = jnp.sum(pm, axis=1, keepdims=True)
        lb = jnp.sum(pm * counts, axis=1, keepdims=True)
        lb_out_ref[...] = lb * (jnp.float32(E) / jnp.float32(A)) / ssum


def _run_mid(xf, sc, wo, ln2_w, ln2_b, wg_pad, gb_pad):
    return pl.pallas_call(
        _mid_body,
        grid=(NTB,),
        in_specs=[
            pl.BlockSpec((TB, D), lambda i: (i, 0)),
            pl.BlockSpec((TB, D), lambda i: (i, 0)),
            pl.BlockSpec((D, D), lambda i: (0, 0)),
            pl.BlockSpec((1, D), lambda i: (0, 0)),
            pl.BlockSpec((1, D), lambda i: (0, 0)),
            pl.BlockSpec((D, 128), lambda i: (0, 0)),
            pl.BlockSpec((1, 128), lambda i: (0, 0)),
        ],
        out_specs=[
            pl.BlockSpec((TB, D), lambda i: (i, 0)),
            pl.BlockSpec((TB, D), lambda i: (i, 0)),
            pl.BlockSpec((TB, 8), lambda i: (i, 0)),
            pl.BlockSpec((T, 1), lambda i: (0, 0)),
            pl.BlockSpec((T, 1), lambda i: (0, 0)),
            pl.BlockSpec((1, NBLK), lambda i: (0, 0)),
            pl.BlockSpec((1, 1), lambda i: (0, 0)),
            pl.BlockSpec((1, 1), lambda i: (0, 0)),
        ],
        out_shape=[
            jax.ShapeDtypeStruct((T, D), jnp.float32),
            jax.ShapeDtypeStruct((T, D), jnp.float32),
            jax.ShapeDtypeStruct((T, 8), jnp.float32),
            jax.ShapeDtypeStruct((T, 1), jnp.int32),
            jax.ShapeDtypeStruct((T, 1), jnp.int32),
            jax.ShapeDtypeStruct((1, NBLK), jnp.int32),
            jax.ShapeDtypeStruct((1, 1), jnp.int32),
            jax.ShapeDtypeStruct((1, 1), jnp.float32),
        ],
        scratch_shapes=[
            pltpu.VMEM((T, 1), jnp.int32),
            pltpu.VMEM((T, 1), jnp.int32),
            pltpu.VMEM((1, 128), jnp.float32),
        ],
    )(xf, sc, wo, ln2_w, ln2_b, wg_pad, gb_pad)


def _sc_mesh():
    return plsc.VectorSubcoreMesh(core_axis_name="c", subcore_axis_name="s")


def _sc_wid():
    return lax.axis_index("s") * 2 + lax.axis_index("c")


def _sc_dispatch(p0, p1, h2):
    tpw = T // NW

    @functools.partial(
        pl.kernel, mesh=_sc_mesh(),
        out_type=jax.ShapeDtypeStruct((P, D), jnp.float32),
        scratch_types=[
            pltpu.VMEM((tpw,), jnp.int32),
            pltpu.VMEM((tpw,), jnp.int32),
            pltpu.VMEM((tpw, D), jnp.float32),
            pltpu.SemaphoreType.DMA,
        ],
    )
    def k(p0_hbm, p1_hbm, h2_hbm, x_hbm, i0_v, i1_v, rows_v, sem):
        base = _sc_wid() * tpw
        pltpu.sync_copy(p0_hbm.at[pl.ds(base, tpw)], i0_v)
        pltpu.sync_copy(p1_hbm.at[pl.ds(base, tpw)], i1_v)
        pltpu.sync_copy(h2_hbm.at[pl.ds(base, tpw)], rows_v)
        c0 = pltpu.async_copy(rows_v, x_hbm.at[i0_v], sem)
        c1 = pltpu.async_copy(rows_v, x_hbm.at[i1_v], sem)
        c0.wait()
        c1.wait()

    return k(p0, p1, h2)


def _sc_combine_gather(p0, p1, y):
    tpw = T // NW

    @functools.partial(
        pl.kernel, mesh=_sc_mesh(),
        out_type=[
            jax.ShapeDtypeStruct((T, D), jnp.float32),
            jax.ShapeDtypeStruct((T, D), jnp.float32),
        ],
        scratch_types=[
            pltpu.VMEM((tpw,), jnp.int32),
            pltpu.VMEM((tpw,), jnp.int32),
            pltpu.VMEM((tpw, D), jnp.float32),
            pltpu.VMEM((tpw, D), jnp.float32),
            pltpu.SemaphoreType.DMA,
        ],
    )
    def k(p0_hbm, p1_hbm, y_hbm, y0_hbm, y1_hbm, i0_v, i1_v, r0_v, r1_v, sem):
        base = _sc_wid() * tpw
        pltpu.sync_copy(p0_hbm.at[pl.ds(base, tpw)], i0_v)
        pltpu.sync_copy(p1_hbm.at[pl.ds(base, tpw)], i1_v)
        c0 = pltpu.async_copy(y_hbm.at[i0_v], r0_v, sem)
        c1 = pltpu.async_copy(y_hbm.at[i1_v], r1_v, sem)
        c0.wait()
        c1.wait()
        pltpu.sync_copy(r0_v, y0_hbm.at[pl.ds(base, tpw)])
        pltpu.sync_copy(r1_v, y1_hbm.at[pl.ds(base, tpw)])

    return k(p0, p1, y)


def _ffn_body(bexp_ref, nact_ref, x_ref, w1_ref, w2_ref, o_ref):
    i = pl.program_id(0)

    @pl.when(i < nact_ref[0])
    def _():
        xb = x_ref[...].astype(jnp.bfloat16)
        pre = jnp.dot(xb, w1_ref[0].astype(jnp.bfloat16),
                      preferred_element_type=jnp.float32)
        x1 = pre[:, :FH]
        x2 = pre[:, FH:]
        act = x1 * (1.0 / (1.0 + jnp.exp(-x1))) * x2
        o_ref[...] = jnp.dot(act.astype(jnp.bfloat16),
                             w2_ref[0].astype(jnp.bfloat16),
                             preferred_element_type=jnp.float32)


def _run_ffn(bexp, nact, xs, w1, w2):
    def wexp(i, b, n):
        return b[jnp.minimum(i, n[0] - 1)]

    grid_spec = pltpu.PrefetchScalarGridSpec(
        num_scalar_prefetch=2,
        grid=(NBLK,),
        in_specs=[
            pl.BlockSpec((BLK, D), lambda i, b, n: (jnp.minimum(i, n[0] - 1), 0)),
            pl.BlockSpec((1, D, 2 * FH), lambda i, b, n: (wexp(i, b, n), 0, 0)),
            pl.BlockSpec((1, FH, D), lambda i, b, n: (wexp(i, b, n), 0, 0)),
        ],
        out_specs=pl.BlockSpec((BLK, D),
                               lambda i, b, n: (jnp.minimum(i, n[0] - 1), 0)),
    )
    return pl.pallas_call(
        _ffn_body,
        grid_spec=grid_spec,
        out_shape=jax.ShapeDtypeStruct((P, D), jnp.float32),
    )(bexp, nact, xs, w1, w2)


def _comb_body(xm_ref, y0_ref, y1_ref, g0_ref, g1_ref, o_ref):
    o_ref[...] = (xm_ref[...]
                  + _b16(g0_ref[...]) * _b16(y0_ref[...])
                  + _b16(g1_ref[...]) * _b16(y1_ref[...]))


def _run_comb(xm, y0, y1, g0, g1):
    return pl.pallas_call(
        _comb_body,
        grid=(NTB,),
        in_specs=[
            pl.BlockSpec((TB, D), lambda i: (i, 0)),
            pl.BlockSpec((TB, D), lambda i: (i, 0)),
            pl.BlockSpec((TB, D), lambda i: (i, 0)),
            pl.BlockSpec((TB, 1), lambda i: (i, 0)),
            pl.BlockSpec((TB, 1), lambda i: (i, 0)),
        ],
        out_specs=pl.BlockSpec((TB, D), lambda i: (i, 0)),
        out_shape=jax.ShapeDtypeStruct((T, D), jnp.float32),
    )(xm, y0, y1, g0, g1)


def kernel(x, ln1_w, ln1_b, ln2_w, ln2_b, Wqkv, Wo, Wg, expert_biases, W1, W2):
    xf = x.reshape(T, D)
    theta = 1.0 / (10000.0 ** (jnp.arange(0, HD, 2, dtype=jnp.float32) / HD))
    ang = jnp.arange(T, dtype=jnp.float32)[:, None] * theta[None, :]
    cos2 = jnp.tile(jnp.repeat(jnp.cos(ang), 2, axis=1), (1, NH))
    sin2 = jnp.tile(jnp.repeat(jnp.sin(ang), 2, axis=1), (1, NH))

    jd = jnp.arange(D)
    rg = (jd[:, None] // HD == jnp.arange(128)[None, :]).astype(jnp.float32)
    rb = (jnp.arange(128)[:, None] == jd[None, :] // HD).astype(jnp.float32)
    rt = (jnp.arange(HD)[:, None] == jd[None, :] % HD).astype(jnp.float32)

    ao_t = _run_attn(xf, ln1_w.reshape(1, D), ln1_b.reshape(1, D),
                     Wqkv, cos2, sin2, rg, rb, rt)
    sc = ao_t.reshape(T, D)

    wg_pad = jnp.zeros((D, 128), jnp.float32).at[:, :E].set(Wg)
    gb_pad = jnp.full((1, 128), NEG, jnp.float32).at[0, :E].set(expert_biases)
    xm, h2, gate8, p0c, p1c, bexp_row, nact, lb = _run_mid(
        xf, sc, Wo, ln2_w.reshape(1, D), ln2_b.reshape(1, D), wg_pad, gb_pad)

    p0 = p0c.reshape(T)
    p1 = p1c.reshape(T)
    xs = _sc_dispatch(p0, p1, h2)
    ys = _run_ffn(bexp_row.reshape(NBLK), nact.reshape(1), xs, W1, W2)
    y0, y1 = _sc_combine_gather(p0, p1, ys)
    out = _run_comb(xm, y0, y1, gate8[:, 0:1], gate8[:, 1:2])
    return (out.reshape(1, T, D), lb[0, 0])


# speedup vs baseline: 2.2978x; 1.0020x over previous
import functools

import jax
import jax.numpy as jnp
from jax import lax
from jax.experimental import pallas as pl
from jax.experimental.pallas import tpu as pltpu
from jax.experimental.pallas import tpu_sc as plsc

D = 768
NH = 12
HD = 64
HH = HD // 2
FH = 512
E = 16
T = 2048
TB = 256
NTB = T // TB
A = T * 2
BLK = 256
NBLK = 32
P = NBLK * BLK
NW = 32
NEG = -1e30


def _b16(a):
    return a.astype(jnp.bfloat16).astype(jnp.float32)


def _attn_body(x_ref, w_ref, b_ref, wqkv_ref, cos_ref, sin_ref,
               rg_ref, rb_ref, rt_ref, out_ref):
    xb = x_ref[...]
    mu = jnp.mean(xb, axis=1, keepdims=True)
    xc = xb - mu
    var = jnp.mean(xc * xc, axis=1, keepdims=True)
    h = xc * lax.rsqrt(var + 1e-5) * w_ref[...] + b_ref[...]
    qkv = jnp.dot(h.astype(jnp.bfloat16), wqkv_ref[...].astype(jnp.bfloat16),
                  preferred_element_type=jnp.float32)

    cos = cos_ref[...]
    sin = sin_ref[...]
    even = (lax.broadcasted_iota(jnp.int32, (TB, D), 1) % 2) == 0

    def rope(xq):
        left = jnp.concatenate([xq[:, 1:], xq[:, :1]], axis=1)
        right = jnp.concatenate([xq[:, -1:], xq[:, :-1]], axis=1)
        rot = jnp.where(even, -left, right)
        return xq * cos + rot * sin

    q = _b16(rope(qkv[:, 0:D]))
    k = _b16(rope(qkv[:, D:2 * D]))
    rg = rg_ref[...].astype(jnp.bfloat16)
    rb = rb_ref[...]
    del rt_ref
    scale = 1.0 / 8.0
    s = []
    for m in range(NH):
        km = jnp.concatenate([k[:, m * HD:(m + 1) * HD]] * NH, axis=1)
        p = q * km
        hi = p.astype(jnp.bfloat16)
        lo = (p - hi.astype(jnp.float32)).astype(jnp.bfloat16)
        sm = (jnp.dot(hi, rg, preferred_element_type=jnp.float32)
              + jnp.dot(lo, rg, preferred_element_type=jnp.float32))
        s.append(sm * scale)
    mx = s[0]
    for m in range(1, NH):
        mx = jnp.maximum(mx, s[m])
    es = [jnp.exp(t_ - mx) for t_ in s]
    den = es[0]
    for m in range(1, NH):
        den = den + es[m]
    acc = None
    for m in range(NH):
        vm = _b16(qkv[:, 2 * D + m * HD: 2 * D + (m + 1) * HD])
        vt = jnp.concatenate([vm] * NH, axis=1)
        ab = jnp.dot(es[m] / den, rb, preferred_element_type=jnp.float32)
        c = ab * vt
        acc = c if acc is None else acc + c
    for n in range(NH):
        out_ref[n, :, :] = acc[:, n * HD:(n + 1) * HD]


def _run_attn(xf, ln1_w, ln1_b, wqkv, cosb, sinb, rg, rb, rt):
    return pl.pallas_call(
        _attn_body,
        grid=(NTB,),
        in_specs=[
            pl.BlockSpec((TB, D), lambda i: (i, 0)),
            pl.BlockSpec((1, D), lambda i: (0, 0)),
            pl.BlockSpec((1, D), lambda i: (0, 0)),
            pl.BlockSpec((D, 3 * D), lambda i: (0, 0)),
            pl.BlockSpec((TB, D), lambda i: (i, 0)),
            pl.BlockSpec((TB, D), lambda i: (i, 0)),
            pl.BlockSpec((D, 128), lambda i: (0, 0)),
            pl.BlockSpec((128, D), lambda i: (0, 0)),
            pl.BlockSpec((HD, D), lambda i: (0, 0)),
        ],
        out_specs=pl.BlockSpec((NH, TB, HD), lambda i: (0, i, 0)),
        out_shape=jax.ShapeDtypeStruct((NH, T, HD), jnp.float32),
    )(xf, ln1_w, ln1_b, wqkv, cosb, sinb, rg, rb, rt)


def _mid_body(x_ref, sc_ref, wo_ref, lw_ref, lb_ref, wg_ref, gb_ref,
              xm_ref, h2b_ref, gate_ref, p0_ref, p1_ref, bexp_ref, nact_ref,
              lb_out_ref, i1_scr, i2_scr, pp_scr):
    i = pl.program_id(0)
    xm = x_ref[...] + jnp.dot(sc_ref[...].astype(jnp.bfloat16),
                              wo_ref[...].astype(jnp.bfloat16),
                              preferred_element_type=jnp.float32)
    xm_ref[...] = xm
    mu = jnp.mean(xm, axis=1, keepdims=True)
    xc = xm - mu
    var = jnp.mean(xc * xc, axis=1, keepdims=True)
    h2 = xc * lax.rsqrt(var + 1e-5) * lw_ref[...] + lb_ref[...]
    h2b_ref[...] = h2
    g = jnp.dot(h2.astype(jnp.bfloat16), wg_ref[...].astype(jnp.bfloat16),
                preferred_element_type=jnp.float32) + gb_ref[...]
    iota = lax.broadcasted_iota(jnp.int32, g.shape, 1)
    big = jnp.int32(10**9)
    v1 = jnp.max(g, axis=1, keepdims=True)
    i1 = jnp.min(jnp.where(g == v1, iota, big), axis=1, keepdims=True)
    gm = jnp.where(iota == i1, NEG, g)
    v2 = jnp.max(gm, axis=1, keepdims=True)
    i2 = jnp.min(jnp.where(gm == v2, iota, big), axis=1, keepdims=True)
    ex = jnp.exp(v2 - v1)
    den2 = 1.0 + ex
    g1 = 1.0 / den2
    g2 = ex / den2
    zf = jnp.zeros_like(g1)
    gate_ref[...] = jnp.concatenate([g1, g2, zf, zf, zf, zf, zf, zf], axis=1)
    i1_scr[pl.ds(i * TB, TB), :] = i1
    i2_scr[pl.ds(i * TB, TB), :] = i2
    p = jnp.exp(g - v1)
    p = p / jnp.sum(p, axis=1, keepdims=True)
    psum = jnp.sum(p, axis=0, keepdims=True)

    @pl.when(i == 0)
    def _():
        pp_scr[...] = psum

    @pl.when(i > 0)
    def _():
        pp_scr[...] = pp_scr[...] + psum

    @pl.when(i == NTB - 1)
    def _():
        lane = lax.broadcasted_iota(jnp.int32, (T, E), 1)
        i1a = i1_scr[...]
        i2a = i2_scr[...]
        eq0 = (jnp.broadcast_to(i1a, (T, E)) == lane).astype(jnp.float32)
        eq1 = (jnp.broadcast_to(i2a, (T, E)) == lane).astype(jnp.float32)
        both = eq0 + eq1
        c = both
        s = 1
        while s < T:
            c = c + jnp.concatenate(
                [jnp.zeros((s, E), jnp.float32), c[:T - s]], axis=0)
            s *= 2
        excl = c - both
        counts = jnp.sum(both, axis=0, keepdims=True)
        nb = (counts.astype(jnp.int32) + (BLK - 1)) // BLK
        z = nb
        for s in (1, 2, 4, 8):
            z = z + jnp.concatenate(
                [jnp.zeros((1, s), jnp.int32), z[:, :E - s]], axis=1)
        off = z - nb
        nact_ref[...] = z[:, E - 1:E]
        slotbase = jnp.broadcast_to((off * BLK).astype(jnp.float32), (T, E))
        p0f = jnp.sum(eq0 * (slotbase + excl), axis=1, keepdims=True)
        p1f = jnp.sum(eq1 * (slotbase + excl + eq0), axis=1, keepdims=True)
        p0_ref[...] = p0f.astype(jnp.int32)
        p1_ref[...] = p1f.astype(jnp.int32)
        biota = lax.broadcasted_iota(jnp.int32, (1, NBLK), 1)
        acc = jnp.full((1, NBLK), -1, jnp.int32)
        for e in range(E):
            acc = acc + (jnp.broadcast_to(off[:, e:e + 1], (1, NBLK))
                         <= biota).astype(jnp.int32)
        bexp_ref[...] = acc
        pm = pp_scr[...][:, :E]
        ssum = jnp.sum(pm, axis=1, keepdims=True)
        lb = jnp.sum(pm * counts, axis=1, keepdims=True)
        lb_out_ref[...] = lb * (jnp.float32(E) / jnp.float32(A)) / ssum


def _run_mid(xf, sc, wo, ln2_w, ln2_b, wg_pad, gb_pad):
    return pl.pallas_call(
        _mid_body,
        grid=(NTB,),
        in_specs=[
            pl.BlockSpec((TB, D), lambda i: (i, 0)),
            pl.BlockSpec((TB, D), lambda i: (i, 0)),
            pl.BlockSpec((D, D), lambda i: (0, 0)),
            pl.BlockSpec((1, D), lambda i: (0, 0)),
            pl.BlockSpec((1, D), lambda i: (0, 0)),
            pl.BlockSpec((D, 128), lambda i: (0, 0)),
            pl.BlockSpec((1, 128), lambda i: (0, 0)),
        ],
        out_specs=[
            pl.BlockSpec((TB, D), lambda i: (i, 0)),
            pl.BlockSpec((TB, D), lambda i: (i, 0)),
            pl.BlockSpec((TB, 8), lambda i: (i, 0)),
            pl.BlockSpec((T, 1), lambda i: (0, 0)),
            pl.BlockSpec((T, 1), lambda i: (0, 0)),
            pl.BlockSpec((1, NBLK), lambda i: (0, 0)),
            pl.BlockSpec((1, 1), lambda i: (0, 0)),
            pl.BlockSpec((1, 1), lambda i: (0, 0)),
        ],
        out_shape=[
            jax.ShapeDtypeStruct((T, D), jnp.float32),
            jax.ShapeDtypeStruct((T, D), jnp.float32),
            jax.ShapeDtypeStruct((T, 8), jnp.float32),
            jax.ShapeDtypeStruct((T, 1), jnp.int32),
            jax.ShapeDtypeStruct((T, 1), jnp.int32),
            jax.ShapeDtypeStruct((1, NBLK), jnp.int32),
            jax.ShapeDtypeStruct((1, 1), jnp.int32),
            jax.ShapeDtypeStruct((1, 1), jnp.float32),
        ],
        scratch_shapes=[
            pltpu.VMEM((T, 1), jnp.int32),
            pltpu.VMEM((T, 1), jnp.int32),
            pltpu.VMEM((1, 128), jnp.float32),
        ],
    )(xf, sc, wo, ln2_w, ln2_b, wg_pad, gb_pad)


def _sc_mesh():
    return plsc.VectorSubcoreMesh(core_axis_name="c", subcore_axis_name="s")


def _sc_wid():
    return lax.axis_index("s") * 2 + lax.axis_index("c")


def _sc_dispatch(p0, p1, h2):
    tpw = T // NW

    @functools.partial(
        pl.kernel, mesh=_sc_mesh(),
        out_type=jax.ShapeDtypeStruct((P, D), jnp.float32),
        scratch_types=[
            pltpu.VMEM((tpw,), jnp.int32),
            pltpu.VMEM((tpw,), jnp.int32),
            pltpu.VMEM((tpw, D), jnp.float32),
            pltpu.SemaphoreType.DMA,
        ],
    )
    def k(p0_hbm, p1_hbm, h2_hbm, x_hbm, i0_v, i1_v, rows_v, sem):
        base = _sc_wid() * tpw
        cr = pltpu.async_copy(h2_hbm.at[pl.ds(base, tpw)], rows_v, sem)
        pltpu.sync_copy(p0_hbm.at[pl.ds(base, tpw)], i0_v)
        pltpu.sync_copy(p1_hbm.at[pl.ds(base, tpw)], i1_v)
        cr.wait()
        c0 = pltpu.async_copy(rows_v, x_hbm.at[i0_v], sem)
        c1 = pltpu.async_copy(rows_v, x_hbm.at[i1_v], sem)
        c0.wait()
        c1.wait()

    return k(p0, p1, h2)


def _sc_combine_gather(p0, p1, y):
    tpw = T // NW

    @functools.partial(
        pl.kernel, mesh=_sc_mesh(),
        out_type=[
            jax.ShapeDtypeStruct((T, D), jnp.float32),
            jax.ShapeDtypeStruct((T, D), jnp.float32),
        ],
        scratch_types=[
            pltpu.VMEM((tpw,), jnp.int32),
            pltpu.VMEM((tpw,), jnp.int32),
            pltpu.VMEM((tpw, D), jnp.float32),
            pltpu.VMEM((tpw, D), jnp.float32),
            pltpu.SemaphoreType.DMA,
        ],
    )
    def k(p0_hbm, p1_hbm, y_hbm, y0_hbm, y1_hbm, i0_v, i1_v, r0_v, r1_v, sem):
        base = _sc_wid() * tpw
        pltpu.sync_copy(p0_hbm.at[pl.ds(base, tpw)], i0_v)
        pltpu.sync_copy(p1_hbm.at[pl.ds(base, tpw)], i1_v)
        c0 = pltpu.async_copy(y_hbm.at[i0_v], r0_v, sem)
        c1 = pltpu.async_copy(y_hbm.at[i1_v], r1_v, sem)
        c0.wait()
        pltpu.sync_copy(r0_v, y0_hbm.at[pl.ds(base, tpw)])
        c1.wait()
        pltpu.sync_copy(r1_v, y1_hbm.at[pl.ds(base, tpw)])

    return k(p0, p1, y)


def _ffn_body(bexp_ref, nact_ref, x_ref, w1_ref, w2_ref, o_ref):
    i = pl.program_id(0)

    @pl.when(i < nact_ref[0])
    def _():
        xb = x_ref[...].astype(jnp.bfloat16)
        pre = jnp.dot(xb, w1_ref[0].astype(jnp.bfloat16),
                      preferred_element_type=jnp.float32)
        x1 = pre[:, :FH]
        x2 = pre[:, FH:]
        act = x1 * (1.0 / (1.0 + jnp.exp(-x1))) * x2
        o_ref[...] = jnp.dot(act.astype(jnp.bfloat16),
                             w2_ref[0].astype(jnp.bfloat16),
                             preferred_element_type=jnp.float32)


def _run_ffn(bexp, nact, xs, w1, w2):
    def wexp(i, b, n):
        return b[jnp.minimum(i, n[0] - 1)]

    grid_spec = pltpu.PrefetchScalarGridSpec(
        num_scalar_prefetch=2,
        grid=(NBLK,),
        in_specs=[
            pl.BlockSpec((BLK, D), lambda i, b, n: (jnp.minimum(i, n[0] - 1), 0)),
            pl.BlockSpec((1, D, 2 * FH), lambda i, b, n: (wexp(i, b, n), 0, 0)),
            pl.BlockSpec((1, FH, D), lambda i, b, n: (wexp(i, b, n), 0, 0)),
        ],
        out_specs=pl.BlockSpec((BLK, D),
                               lambda i, b, n: (jnp.minimum(i, n[0] - 1), 0)),
    )
    return pl.pallas_call(
        _ffn_body,
        grid_spec=grid_spec,
        out_shape=jax.ShapeDtypeStruct((P, D), jnp.float32),
    )(bexp, nact, xs, w1, w2)


def _comb_body(xm_ref, y0_ref, y1_ref, g0_ref, g1_ref, o_ref):
    o_ref[...] = (xm_ref[...]
                  + _b16(g0_ref[...]) * _b16(y0_ref[...])
                  + _b16(g1_ref[...]) * _b16(y1_ref[...]))


def _run_comb(xm, y0, y1, g0, g1):
    return pl.pallas_call(
        _comb_body,
        grid=(NTB,),
        in_specs=[
            pl.BlockSpec((TB, D), lambda i: (i, 0)),
            pl.BlockSpec((TB, D), lambda i: (i, 0)),
            pl.BlockSpec((TB, D), lambda i: (i, 0)),
            pl.BlockSpec((TB, 1), lambda i: (i, 0)),
            pl.BlockSpec((TB, 1), lambda i: (i, 0)),
        ],
        out_specs=pl.BlockSpec((TB, D), lambda i: (i, 0)),
        out_shape=jax.ShapeDtypeStruct((T, D), jnp.float32),
    )(xm, y0, y1, g0, g1)


def kernel(x, ln1_w, ln1_b, ln2_w, ln2_b, Wqkv, Wo, Wg, expert_biases, W1, W2):
    xf = x.reshape(T, D)
    theta = 1.0 / (10000.0 ** (jnp.arange(0, HD, 2, dtype=jnp.float32) / HD))
    ang = jnp.arange(T, dtype=jnp.float32)[:, None] * theta[None, :]
    cos2 = jnp.tile(jnp.repeat(jnp.cos(ang), 2, axis=1), (1, NH))
    sin2 = jnp.tile(jnp.repeat(jnp.sin(ang), 2, axis=1), (1, NH))

    jd = jnp.arange(D)
    rg = (jd[:, None] // HD == jnp.arange(128)[None, :]).astype(jnp.float32)
    rb = (jnp.arange(128)[:, None] == jd[None, :] // HD).astype(jnp.float32)
    rt = (jnp.arange(HD)[:, None] == jd[None, :] % HD).astype(jnp.float32)

    ao_t = _run_attn(xf, ln1_w.reshape(1, D), ln1_b.reshape(1, D),
                     Wqkv, cos2, sin2, rg, rb, rt)
    sc = ao_t.reshape(T, D)

    wg_pad = jnp.zeros((D, 128), jnp.float32).at[:, :E].set(Wg)
    gb_pad = jnp.full((1, 128), NEG, jnp.float32).at[0, :E].set(expert_biases)
    xm, h2, gate8, p0c, p1c, bexp_row, nact, lb = _run_mid(
        xf, sc, Wo, ln2_w.reshape(1, D), ln2_b.reshape(1, D), wg_pad, gb_pad)

    p0 = p0c.reshape(T)
    p1 = p1c.reshape(T)
    xs = _sc_dispatch(p0, p1, h2)
    ys = _run_ffn(bexp_row.reshape(NBLK), nact.reshape(1), xs, W1, W2)
    y0, y1 = _sc_combine_gather(p0, p1, ys)
    out = _run_comb(xm, y0, y1, gate8[:, 0:1], gate8[:, 1:2])
    return (out.reshape(1, T, D), lb[0, 0])


# speedup vs baseline: 2.3057x; 1.0035x over previous
import functools

import jax
import jax.numpy as jnp
from jax import lax
from jax.experimental import pallas as pl
from jax.experimental.pallas import tpu as pltpu
from jax.experimental.pallas import tpu_sc as plsc

D = 768
NH = 12
HD = 64
HH = HD // 2
FH = 512
E = 16
T = 2048
TB = 256
NTB = T // TB
A = T * 2
BLK = 256
NBLK = 32
P = NBLK * BLK
NW = 32
NEG = -1e30


def _b16(a):
    return a.astype(jnp.bfloat16).astype(jnp.float32)


def _attn_body(x_ref, w_ref, b_ref, wqkv_ref, cos_ref, sin_ref,
               rg_ref, rb_ref, out_ref):
    xb = x_ref[...]
    mu = jnp.mean(xb, axis=1, keepdims=True)
    xc = xb - mu
    var = jnp.mean(xc * xc, axis=1, keepdims=True)
    h = xc * lax.rsqrt(var + 1e-5) * w_ref[...] + b_ref[...]
    qkv = jnp.dot(h.astype(jnp.bfloat16), wqkv_ref[...].astype(jnp.bfloat16),
                  preferred_element_type=jnp.float32)

    cos = cos_ref[...]
    sin = sin_ref[...]
    even = (lax.broadcasted_iota(jnp.int32, (TB, D), 1) % 2) == 0

    def rope(xq):
        left = jnp.concatenate([xq[:, 1:], xq[:, :1]], axis=1)
        right = jnp.concatenate([xq[:, -1:], xq[:, :-1]], axis=1)
        rot = jnp.where(even, -left, right)
        return xq * cos + rot * sin

    q = _b16(rope(qkv[:, 0:D]))
    k = _b16(rope(qkv[:, D:2 * D]))
    rg = rg_ref[...].astype(jnp.bfloat16)
    rb = rb_ref[...]
    scale = 1.0 / 8.0
    s = []
    for m in range(NH):
        km = jnp.concatenate([k[:, m * HD:(m + 1) * HD]] * NH, axis=1)
        p = q * km
        hi = p.astype(jnp.bfloat16)
        lo = (p - hi.astype(jnp.float32)).astype(jnp.bfloat16)
        sm = (jnp.dot(hi, rg, preferred_element_type=jnp.float32)
              + jnp.dot(lo, rg, preferred_element_type=jnp.float32))
        s.append(sm * scale)
    mx = s[0]
    for m in range(1, NH):
        mx = jnp.maximum(mx, s[m])
    es = [jnp.exp(t_ - mx) for t_ in s]
    den = es[0]
    for m in range(1, NH):
        den = den + es[m]
    acc = None
    for m in range(NH):
        vm = _b16(qkv[:, 2 * D + m * HD: 2 * D + (m + 1) * HD])
        vt = jnp.concatenate([vm] * NH, axis=1)
        ab = jnp.dot(es[m] / den, rb, preferred_element_type=jnp.float32)
        c = ab * vt
        acc = c if acc is None else acc + c
    for n in range(NH):
        out_ref[n, :, :] = acc[:, n * HD:(n + 1) * HD]


def _run_attn(xf, ln1_w, ln1_b, wqkv, cosb, sinb, rg, rb):
    return pl.pallas_call(
        _attn_body,
        grid=(NTB,),
        in_specs=[
            pl.BlockSpec((TB, D), lambda i: (i, 0)),
            pl.BlockSpec((1, D), lambda i: (0, 0)),
            pl.BlockSpec((1, D), lambda i: (0, 0)),
            pl.BlockSpec((D, 3 * D), lambda i: (0, 0)),
            pl.BlockSpec((TB, D), lambda i: (i, 0)),
            pl.BlockSpec((TB, D), lambda i: (i, 0)),
            pl.BlockSpec((D, 128), lambda i: (0, 0)),
            pl.BlockSpec((128, D), lambda i: (0, 0)),
        ],
        out_specs=pl.BlockSpec((NH, TB, HD), lambda i: (0, i, 0)),
        out_shape=jax.ShapeDtypeStruct((NH, T, HD), jnp.float32),
    )(xf, ln1_w, ln1_b, wqkv, cosb, sinb, rg, rb)


def _mid_body(x_ref, sc_ref, wo_ref, lw_ref, lb_ref, wg_ref, gb_ref,
              xm_ref, h2b_ref, gate_ref, p0_ref, p1_ref, bexp_ref, nact_ref,
              lb_out_ref, i1_scr, i2_scr, pp_scr):
    i = pl.program_id(0)
    xm = x_ref[...] + jnp.dot(sc_ref[...].astype(jnp.bfloat16),
                              wo_ref[...].astype(jnp.bfloat16),
                              preferred_element_type=jnp.float32)
    xm_ref[...] = xm
    mu = jnp.mean(xm, axis=1, keepdims=True)
    xc = xm - mu
    var = jnp.mean(xc * xc, axis=1, keepdims=True)
    h2 = xc * lax.rsqrt(var + 1e-5) * lw_ref[...] + lb_ref[...]
    h2b_ref[...] = h2
    g = jnp.dot(h2.astype(jnp.bfloat16), wg_ref[...].astype(jnp.bfloat16),
                preferred_element_type=jnp.float32) + gb_ref[...]
    iota = lax.broadcasted_iota(jnp.int32, g.shape, 1)
    big = jnp.int32(10**9)
    v1 = jnp.max(g, axis=1, keepdims=True)
    i1 = jnp.min(jnp.where(g == v1, iota, big), axis=1, keepdims=True)
    gm = jnp.where(iota == i1, NEG, g)
    v2 = jnp.max(gm, axis=1, keepdims=True)
    i2 = jnp.min(jnp.where(gm == v2, iota, big), axis=1, keepdims=True)
    ex = jnp.exp(v2 - v1)
    den2 = 1.0 + ex
    g1 = 1.0 / den2
    g2 = ex / den2
    zf = jnp.zeros_like(g1)
    gate_ref[...] = jnp.concatenate([g1, g2, zf, zf, zf, zf, zf, zf], axis=1)
    i1_scr[pl.ds(i * TB, TB), :] = i1
    i2_scr[pl.ds(i * TB, TB), :] = i2
    p = jnp.exp(g - v1)
    p = p / jnp.sum(p, axis=1, keepdims=True)
    psum = jnp.sum(p, axis=0, keepdims=True)

    @pl.when(i == 0)
    def _():
        pp_scr[...] = psum

    @pl.when(i > 0)
    def _():
        pp_scr[...] = pp_scr[...] + psum

    @pl.when(i == NTB - 1)
    def _():
        lane = lax.broadcasted_iota(jnp.int32, (T, E), 1)
        i1a = i1_scr[...]
        i2a = i2_scr[...]
        eq0 = (jnp.broadcast_to(i1a, (T, E)) == lane).astype(jnp.float32)
        eq1 = (jnp.broadcast_to(i2a, (T, E)) == lane).astype(jnp.float32)
        both = eq0 + eq1
        c = both
        s = 1
        while s < T:
            c = c + jnp.concatenate(
                [jnp.zeros((s, E), jnp.float32), c[:T - s]], axis=0)
            s *= 2
        excl = c - both
        counts = jnp.sum(both, axis=0, keepdims=True)
        nb = (counts.astype(jnp.int32) + (BLK - 1)) // BLK
        z = nb
        for s in (1, 2, 4, 8):
            z = z + jnp.concatenate(
                [jnp.zeros((1, s), jnp.int32), z[:, :E - s]], axis=1)
        off = z - nb
        nact_ref[...] = z[:, E - 1:E]
        slotbase = jnp.broadcast_to((off * BLK).astype(jnp.float32), (T, E))
        p0f = jnp.sum(eq0 * (slotbase + excl), axis=1, keepdims=True)
        p1f = jnp.sum(eq1 * (slotbase + excl + eq0), axis=1, keepdims=True)
        p0_ref[...] = p0f.astype(jnp.int32)
        p1_ref[...] = p1f.astype(jnp.int32)
        biota = lax.broadcasted_iota(jnp.int32, (1, NBLK), 1)
        acc = jnp.full((1, NBLK), -1, jnp.int32)
        for e in range(E):
            acc = acc + (jnp.broadcast_to(off[:, e:e + 1], (1, NBLK))
                         <= biota).astype(jnp.int32)
        bexp_ref[...] = acc
        pm = pp_scr[...][:, :E]
        ssum = jnp.sum(pm, axis=1, keepdims=True)
        lb = jnp.sum(pm * counts, axis=1, keepdims=True)
        lb_out_ref[...] = lb * (jnp.float32(E) / jnp.float32(A)) / ssum


def _run_mid(xf, sc, wo, ln2_w, ln2_b, wg_pad, gb_pad):
    return pl.pallas_call(
        _mid_body,
        grid=(NTB,),
        in_specs=[
            pl.BlockSpec((TB, D), lambda i: (i, 0)),
            pl.BlockSpec((TB, D), lambda i: (i, 0)),
            pl.BlockSpec((D, D), lambda i: (0, 0)),
            pl.BlockSpec((1, D), lambda i: (0, 0)),
            pl.BlockSpec((1, D), lambda i: (0, 0)),
            pl.BlockSpec((D, 128), lambda i: (0, 0)),
            pl.BlockSpec((1, 128), lambda i: (0, 0)),
        ],
        out_specs=[
            pl.BlockSpec((TB, D), lambda i: (i, 0)),
            pl.BlockSpec((TB, D), lambda i: (i, 0)),
            pl.BlockSpec((TB, 8), lambda i: (i, 0)),
            pl.BlockSpec((T, 1), lambda i: (0, 0)),
            pl.BlockSpec((T, 1), lambda i: (0, 0)),
            pl.BlockSpec((1, NBLK), lambda i: (0, 0)),
            pl.BlockSpec((1, 1), lambda i: (0, 0)),
            pl.BlockSpec((1, 1), lambda i: (0, 0)),
        ],
        out_shape=[
            jax.ShapeDtypeStruct((T, D), jnp.float32),
            jax.ShapeDtypeStruct((T, D), jnp.float32),
            jax.ShapeDtypeStruct((T, 8), jnp.float32),
            jax.ShapeDtypeStruct((T, 1), jnp.int32),
            jax.ShapeDtypeStruct((T, 1), jnp.int32),
            jax.ShapeDtypeStruct((1, NBLK), jnp.int32),
            jax.ShapeDtypeStruct((1, 1), jnp.int32),
            jax.ShapeDtypeStruct((1, 1), jnp.float32),
        ],
        scratch_shapes=[
            pltpu.VMEM((T, 1), jnp.int32),
            pltpu.VMEM((T, 1), jnp.int32),
            pltpu.VMEM((1, 128), jnp.float32),
        ],
    )(xf, sc, wo, ln2_w, ln2_b, wg_pad, gb_pad)


def _sc_mesh():
    return plsc.VectorSubcoreMesh(core_axis_name="c", subcore_axis_name="s")


def _sc_wid():
    return lax.axis_index("s") * 2 + lax.axis_index("c")


def _sc_dispatch(p0, p1, h2):
    tpw = T // NW

    @functools.partial(
        pl.kernel, mesh=_sc_mesh(),
        out_type=jax.ShapeDtypeStruct((P, D), jnp.float32),
        scratch_types=[
            pltpu.VMEM((tpw,), jnp.int32),
            pltpu.VMEM((tpw,), jnp.int32),
            pltpu.VMEM((tpw, D), jnp.float32),
            pltpu.SemaphoreType.DMA,
        ],
    )
    def k(p0_hbm, p1_hbm, h2_hbm, x_hbm, i0_v, i1_v, rows_v, sem):
        base = _sc_wid() * tpw
        cr = pltpu.async_copy(h2_hbm.at[pl.ds(base, tpw)], rows_v, sem)
        pltpu.sync_copy(p0_hbm.at[pl.ds(base, tpw)], i0_v)
        pltpu.sync_copy(p1_hbm.at[pl.ds(base, tpw)], i1_v)
        cr.wait()
        c0 = pltpu.async_copy(rows_v, x_hbm.at[i0_v], sem)
        c1 = pltpu.async_copy(rows_v, x_hbm.at[i1_v], sem)
        c0.wait()
        c1.wait()

    return k(p0, p1, h2)


def _sc_combine_gather(p0, p1, y):
    tpw = T // NW

    @functools.partial(
        pl.kernel, mesh=_sc_mesh(),
        out_type=[
            jax.ShapeDtypeStruct((T, D), jnp.float32),
            jax.ShapeDtypeStruct((T, D), jnp.float32),
        ],
        scratch_types=[
            pltpu.VMEM((tpw,), jnp.int32),
            pltpu.VMEM((tpw,), jnp.int32),
            pltpu.VMEM((tpw, D), jnp.float32),
            pltpu.VMEM((tpw, D), jnp.float32),
            pltpu.SemaphoreType.DMA,
        ],
    )
    def k(p0_hbm, p1_hbm, y_hbm, y0_hbm, y1_hbm, i0_v, i1_v, r0_v, r1_v, sem):
        base = _sc_wid() * tpw
        pltpu.sync_copy(p0_hbm.at[pl.ds(base, tpw)], i0_v)
        pltpu.sync_copy(p1_hbm.at[pl.ds(base, tpw)], i1_v)
        c0 = pltpu.async_copy(y_hbm.at[i0_v], r0_v, sem)
        c1 = pltpu.async_copy(y_hbm.at[i1_v], r1_v, sem)
        c0.wait()
        pltpu.sync_copy(r0_v, y0_hbm.at[pl.ds(base, tpw)])
        c1.wait()
        pltpu.sync_copy(r1_v, y1_hbm.at[pl.ds(base, tpw)])

    return k(p0, p1, y)


def _ffn_body(bexp_ref, nact_ref, x_ref, w1_ref, w2_ref, o_ref):
    i = pl.program_id(0)

    @pl.when(i < nact_ref[0])
    def _():
        xb = x_ref[...].astype(jnp.bfloat16)
        pre = jnp.dot(xb, w1_ref[0].astype(jnp.bfloat16),
                      preferred_element_type=jnp.float32)
        x1 = pre[:, :FH]
        x2 = pre[:, FH:]
        act = x1 * (1.0 / (1.0 + jnp.exp(-x1))) * x2
        o_ref[...] = jnp.dot(act.astype(jnp.bfloat16),
                             w2_ref[0].astype(jnp.bfloat16),
                             preferred_element_type=jnp.float32)


def _run_ffn(bexp, nact, xs, w1, w2):
    def wexp(i, b, n):
        return b[jnp.minimum(i, n[0] - 1)]

    grid_spec = pltpu.PrefetchScalarGridSpec(
        num_scalar_prefetch=2,
        grid=(NBLK,),
        in_specs=[
            pl.BlockSpec((BLK, D), lambda i, b, n: (jnp.minimum(i, n[0] - 1), 0)),
            pl.BlockSpec((1, D, 2 * FH), lambda i, b, n: (wexp(i, b, n), 0, 0)),
            pl.BlockSpec((1, FH, D), lambda i, b, n: (wexp(i, b, n), 0, 0)),
        ],
        out_specs=pl.BlockSpec((BLK, D),
                               lambda i, b, n: (jnp.minimum(i, n[0] - 1), 0)),
    )
    return pl.pallas_call(
        _ffn_body,
        grid_spec=grid_spec,
        out_shape=jax.ShapeDtypeStruct((P, D), jnp.float32),
    )(bexp, nact, xs, w1, w2)


def _comb_body(xm_ref, y0_ref, y1_ref, g0_ref, g1_ref, o_ref):
    o_ref[...] = (xm_ref[...]
                  + _b16(g0_ref[...]) * _b16(y0_ref[...])
                  + _b16(g1_ref[...]) * _b16(y1_ref[...]))


def _run_comb(xm, y0, y1, g0, g1):
    return pl.pallas_call(
        _comb_body,
        grid=(NTB,),
        in_specs=[
            pl.BlockSpec((TB, D), lambda i: (i, 0)),
            pl.BlockSpec((TB, D), lambda i: (i, 0)),
            pl.BlockSpec((TB, D), lambda i: (i, 0)),
            pl.BlockSpec((TB, 1), lambda i: (i, 0)),
            pl.BlockSpec((TB, 1), lambda i: (i, 0)),
        ],
        out_specs=pl.BlockSpec((TB, D), lambda i: (i, 0)),
        out_shape=jax.ShapeDtypeStruct((T, D), jnp.float32),
    )(xm, y0, y1, g0, g1)


def kernel(x, ln1_w, ln1_b, ln2_w, ln2_b, Wqkv, Wo, Wg, expert_biases, W1, W2):
    xf = x.reshape(T, D)
    theta = 1.0 / (10000.0 ** (jnp.arange(0, HD, 2, dtype=jnp.float32) / HD))
    ang = jnp.arange(T, dtype=jnp.float32)[:, None] * theta[None, :]
    cos2 = jnp.tile(jnp.repeat(jnp.cos(ang), 2, axis=1), (1, NH))
    sin2 = jnp.tile(jnp.repeat(jnp.sin(ang), 2, axis=1), (1, NH))

    jd = jnp.arange(D)
    rg = (jd[:, None] // HD == jnp.arange(128)[None, :]).astype(jnp.float32)
    rb = (jnp.arange(128)[:, None] == jd[None, :] // HD).astype(jnp.float32)

    ao_t = _run_attn(xf, ln1_w.reshape(1, D), ln1_b.reshape(1, D),
                     Wqkv, cos2, sin2, rg, rb)
    sc = ao_t.reshape(T, D)

    wg_pad = jnp.zeros((D, 128), jnp.float32).at[:, :E].set(Wg)
    gb_pad = jnp.full((1, 128), NEG, jnp.float32).at[0, :E].set(expert_biases)
    xm, h2, gate8, p0c, p1c, bexp_row, nact, lb = _run_mid(
        xf, sc, Wo, ln2_w.reshape(1, D), ln2_b.reshape(1, D), wg_pad, gb_pad)

    p0 = p0c.reshape(T)
    p1 = p1c.reshape(T)
    xs = _sc_dispatch(p0, p1, h2)
    ys = _run_ffn(bexp_row.reshape(NBLK), nact.reshape(1), xs, W1, W2)
    y0, y1 = _sc_combine_gather(p0, p1, ys)
    out = _run_comb(xm, y0, y1, gate8[:, 0:1], gate8[:, 1:2])
    return (out.reshape(1, T, D), lb[0, 0])
